# Initial kernel scaffold; baseline (speedup 1.0000x reference)
import jax
import jax.numpy as jnp
from jax.experimental import pallas as pl


def kernel(input, input_timestamp, diffusion_edge_index, emb, Wg1, bg1, Wg2, bg2, pos_emb, time_emb, Wq, bq, Wk, bk, Wv, bv, Wo, bo, ln1g, ln1b, ln2g, ln2b, Wf1, bf1, Wf2, bf2, Wout, bout):
    raise NotImplementedError("write your pallas kernel here")


# speedup vs baseline: 13.0267x; 13.0267x over previous
import functools

import jax
import jax.numpy as jnp
from jax import lax
from jax.experimental import pallas as pl
from jax.experimental.pallas import tpu as pltpu
from jax.experimental.pallas import tpu_sc as plsc

PAD = 0
NTOKEN = 10000
NINP = 128
POS_DIM = 8
TSTEPS = 8
NHEADS = 8
BATCH = 8
SEQ = 201
NEDGES = 320000

NTP = 10240
WA = 128
WB = 16
NC, NS, NW = 2, 16, 32
EPT = NEDGES // NW
CH = 128
NCHUNK = (EPT + CH - 1) // CH
L = SEQ - 1
D = NINP + POS_DIM
HD = D // NHEADS
NQ = BATCH * L
QPT = NQ * TSTEPS // NW
RPT = NTP // NS

_mesh = functools.partial(plsc.VectorSubcoreMesh,
                          core_axis_name="c", subcore_axis_name="s")


def _deg_body(dst_hbm, out_hbm, dstr, hist):
    cid = lax.axis_index("c")
    sid = lax.axis_index("s")
    wid = cid * NS + sid
    zv = jnp.zeros((16,), jnp.float32)
    ones = jnp.ones((16,), jnp.float32)

    def zero_step(i, _):
        hist[i, :] = zv
        return 0
    lax.fori_loop(0, TSTEPS * NTP // 16, zero_step, 0)

    def t_step(t, _):
        pltpu.sync_copy(dst_hbm.at[pl.ds(t * NEDGES + wid * EPT, EPT)], dstr)

        def e_step(i, _):
            idx = dstr[pl.ds(i * 16, 16)] + t * NTP
            plsc.addupdate_scatter(hist, [idx >> 4, idx & 15], ones)
            return 0
        lax.fori_loop(0, EPT // 16, e_step, 0)
        return 0
    lax.fori_loop(0, TSTEPS, t_step, 0)
    pltpu.sync_copy(hist, out_hbm.at[wid])


def _sc_degree(dst_flat):
    k = pl.kernel(
        _deg_body,
        out_type=jax.ShapeDtypeStruct((NW, TSTEPS * NTP // 16, 16), jnp.float32),
        mesh=_mesh(),
        scratch_types=[
            pltpu.VMEM((EPT,), jnp.int32),
            pltpu.VMEM((TSTEPS * NTP // 16, 16), jnp.float32),
        ],
        compiler_params=pltpu.CompilerParams(needs_layout_passes=False, use_tc_tiling_on_sc=False),
    )
    return k(dst_flat).reshape(NW, TSTEPS * NTP)


HB = EPT // 2
NCH2 = (HB + CH - 1) // CH
SPAD = NCH2 * CH


def _hop_body(has_aux, *args):
    if has_aux:
        (x_hbm, xb_hbm, src_hbm, dst_hbm, out_hbm, outb_hbm,
         srcr, dstr, dsts, rows, rowsb, acc, accb, sem) = args
    else:
        (x_hbm, src_hbm, dst_hbm, out_hbm,
         srcr, dstr, dsts, rows, acc, sem) = args
    cid = lax.axis_index("c")
    sid = lax.axis_index("s")
    wid = cid * NS + sid
    zv = jnp.zeros((16,), jnp.float32)
    iota = jnp.arange(16, dtype=jnp.int32)

    def t_step(t, _):
        def z_step(i, _):
            j = i // (WA // 16)
            k = i % (WA // 16)
            rows[j, pl.ds(k * 16, 16)] = zv
            return 0
        lax.fori_loop(0, CH * (WA // 16), z_step, 0)

        def clr(m, _):
            pltpu.sync_copy(rows, acc.at[pl.ds(sid * RPT + m * CH, CH)])
            return 0
        lax.fori_loop(0, RPT // CH, clr, 0)
        if has_aux:
            def zb_step(i, _):
                rowsb[i, :] = zv
                return 0
            lax.fori_loop(0, CH, zb_step, 0)

            def clrb(m, _):
                pltpu.sync_copy(rowsb, accb.at[pl.ds(sid * RPT + m * CH, CH)])
                return 0
            lax.fori_loop(0, RPT // CH, clrb, 0)

        gpad = t * NTP + NTOKEN + sid * 8
        spad = NTOKEN + sid * 8 + cid

        def batch(b2, _):
            ebase = t * NEDGES + wid * EPT + b2 * HB
            pltpu.sync_copy(src_hbm.at[pl.ds(ebase, HB)],
                            srcr.at[pl.ds(0, HB)])
            pltpu.sync_copy(dst_hbm.at[pl.ds(ebase, HB)],
                            dstr.at[pl.ds(0, HB)])

            def stage(i, _):
                j = i // (CH // 16)
                k = i % (CH // 16)
                pos = i * 16
                valid = (pos + iota) < HB
                sv = srcr[pl.ds(pos, 16)]
                dv = dstr[pl.ds(pos, 16)]
                srcr[pl.ds(pos, 16)] = jnp.where(valid, sv + t * NTP, gpad)
                dsts[j, pl.ds(k * 16, 16)] = jnp.where(valid, dv, spad)
                return 0
            lax.fori_loop(0, NCH2 * (CH // 16), stage, 0)

            def chunk(j, _):
                pltpu.async_copy(x_hbm.at[srcr.at[pl.ds(j * CH, CH)]],
                                 rows, sem).wait()
                pltpu.sync_copy(rows, acc.at[dsts.at[j]], add=True)
                if has_aux:
                    pltpu.async_copy(xb_hbm.at[srcr.at[pl.ds(j * CH, CH)]],
                                     rowsb, sem).wait()
                    pltpu.sync_copy(rowsb, accb.at[dsts.at[j]], add=True)
                return 0
            lax.fori_loop(0, NCH2, chunk, 0)
            return 0
        lax.fori_loop(0, 2, batch, 0)

        plsc.subcore_barrier()

        def wout(m, _):
            r0 = sid * RPT + m * CH
            pltpu.sync_copy(acc.at[pl.ds(r0, CH)],
                            out_hbm.at[cid, t, pl.ds(r0, CH), :])
            return 0
        lax.fori_loop(0, RPT // CH, wout, 0)
        if has_aux:
            def woutb(m, _):
                r0 = sid * RPT + m * CH
                pltpu.sync_copy(accb.at[pl.ds(r0, CH)],
                                outb_hbm.at[cid, t, pl.ds(r0, CH), :])
                return 0
            lax.fori_loop(0, RPT // CH, woutb, 0)

        plsc.subcore_barrier()
        return 0
    lax.fori_loop(0, TSTEPS, t_step, 0)


def _sc_hop(x_tables, src_flat, dst_flat, xb_tables=None):
    has_aux = xb_tables is not None
    out_type = [jax.ShapeDtypeStruct((NC, TSTEPS, NTP, WA), jnp.float32)]
    scratch = [
        pltpu.VMEM((SPAD,), jnp.int32),
        pltpu.VMEM((SPAD,), jnp.int32),
        pltpu.VMEM((NCH2, CH), jnp.int32),
        pltpu.VMEM((CH, WA), jnp.float32),
    ]
    if has_aux:
        out_type.append(jax.ShapeDtypeStruct((NC, TSTEPS, NTP, WB), jnp.float32))
        scratch.append(pltpu.VMEM((CH, WB), jnp.float32))
    scratch.append(pltpu.VMEM_SHARED((NTP, WA), jnp.float32))
    if has_aux:
        scratch.append(pltpu.VMEM_SHARED((NTP, WB), jnp.float32))
    scratch.append(pltpu.SemaphoreType.DMA)
    k = pl.kernel(
        functools.partial(_hop_body, has_aux),
        out_type=out_type if has_aux else out_type[0],
        name=f"sc_hop_aux{int(has_aux)}",
        mesh=_mesh(),
        scratch_types=scratch,
        compiler_params=pltpu.CompilerParams(needs_layout_passes=False,
                                             use_tc_tiling_on_sc=False),
    )
    if has_aux:
        s, sb = k(x_tables, xb_tables, src_flat, dst_flat)
        return (s.reshape(NC, TSTEPS * NTP, WA),
                sb.reshape(NC, TSTEPS * NTP, WB))
    s = k(x_tables, src_flat, dst_flat)
    return s.reshape(NC, TSTEPS * NTP, WA)


def _qgather_body(g_hbm, gb_hbm, qidx_hbm, out_hbm, outb_hbm,
                  qv, rows, rowsb, sem):
    cid = lax.axis_index("c")
    sid = lax.axis_index("s")
    wid = cid * NS + sid
    base = wid * QPT
    pltpu.sync_copy(qidx_hbm.at[pl.ds(base, QPT)], qv)

    def chunk(j, _):
        pltpu.async_copy(g_hbm.at[qv.at[pl.ds(j * 80, 80)]], rows, sem).wait()
        pltpu.sync_copy(rows, out_hbm.at[pl.ds(base + j * 80, 80), :])
        pltpu.async_copy(gb_hbm.at[qv.at[pl.ds(j * 80, 80)]], rowsb, sem).wait()
        pltpu.sync_copy(rowsb, outb_hbm.at[pl.ds(base + j * 80, 80), :])
        return 0
    lax.fori_loop(0, QPT // 80, chunk, 0)


def _sc_qgather(g_tables, gb_tables, qidx):
    k = pl.kernel(
        _qgather_body,
        out_type=[jax.ShapeDtypeStruct((TSTEPS * NQ, WA), jnp.float32),
                  jax.ShapeDtypeStruct((TSTEPS * NQ, WB), jnp.float32)],
        mesh=_mesh(),
        scratch_types=[
            pltpu.VMEM((QPT,), jnp.int32),
            pltpu.VMEM((80, WA), jnp.float32),
            pltpu.VMEM((80, WB), jnp.float32),
            pltpu.SemaphoreType.DMA,
        ],
        compiler_params=pltpu.CompilerParams(needs_layout_passes=False, use_tc_tiling_on_sc=False),
    )
    return k(g_tables, gb_tables, qidx)


_BLK = 640
_NROW = TSTEPS * NTP
_NBLK = _NROW // _BLK


def _tca_body(degp, emb, dinv, x1, x1b):
    deg = jnp.sum(degp[...], axis=0) + 1.0
    dv = lax.rsqrt(deg)[:, None]
    dinv[...] = dv
    x1[...] = dv * emb[...]
    x1b[...] = jnp.concatenate(
        [dv, jnp.zeros((_BLK, WB - 1), jnp.float32)], axis=-1)


def _tc_prep1(deg_partials, embp8):
    return pl.pallas_call(
        _tca_body,
        grid=(_NBLK,),
        in_specs=[
            pl.BlockSpec((NW, _BLK), lambda j: (0, j)),
            pl.BlockSpec((_BLK, NINP), lambda j: (j % (NTP // _BLK), 0)),
        ],
        out_specs=[
            pl.BlockSpec((_BLK, 1), lambda j: (j, 0)),
            pl.BlockSpec((_BLK, WA), lambda j: (j, 0)),
            pl.BlockSpec((_BLK, WB), lambda j: (j, 0)),
        ],
        out_shape=[
            jax.ShapeDtypeStruct((_NROW, 1), jnp.float32),
            jax.ShapeDtypeStruct((_NROW, WA), jnp.float32),
            jax.ShapeDtypeStruct((_NROW, WB), jnp.float32),
        ],
    )(deg_partials, embp8)


def _tcb_body(s1p, s1pb, x1, x1b, dinv, x2, rs):
    s = s1p[0] + s1p[1] + x1[...]
    p = dinv[...] * s
    x2[...] = dinv[...] * p
    sb = s1pb[0] + s1pb[1] + x1b[...]
    rs[...] = dinv[...] * sb[:, 0:1]


def _tc_prep2(s1_partials, s1b_partials, x1, x1b, dinv):
    return pl.pallas_call(
        _tcb_body,
        grid=(_NBLK,),
        in_specs=[
            pl.BlockSpec((NC, _BLK, WA), lambda j: (0, j, 0)),
            pl.BlockSpec((NC, _BLK, WB), lambda j: (0, j, 0)),
            pl.BlockSpec((_BLK, WA), lambda j: (j, 0)),
            pl.BlockSpec((_BLK, WB), lambda j: (j, 0)),
            pl.BlockSpec((_BLK, 1), lambda j: (j, 0)),
        ],
        out_specs=[
            pl.BlockSpec((_BLK, WA), lambda j: (j, 0)),
            pl.BlockSpec((_BLK, 1), lambda j: (j, 0)),
        ],
        out_shape=[
            jax.ShapeDtypeStruct((_NROW, WA), jnp.float32),
            jax.ShapeDtypeStruct((_NROW, 1), jnp.float32),
        ],
    )(s1_partials, s1b_partials, x1, x1b, dinv)


def _tcc_body(s2p, x2, dinv, rs, g, gb):
    s = s2p[0] + s2p[1] + x2[...]
    g[...] = dinv[...] * s
    gb[...] = jnp.concatenate(
        [rs[...], jnp.zeros((_BLK, WB - 1), jnp.float32)], axis=-1)


def _tc_prep3(s2_partials, x2, dinv, rs):
    return pl.pallas_call(
        _tcc_body,
        grid=(_NBLK,),
        in_specs=[
            pl.BlockSpec((NC, _BLK, WA), lambda j: (0, j, 0)),
            pl.BlockSpec((_BLK, WA), lambda j: (j, 0)),
            pl.BlockSpec((_BLK, 1), lambda j: (j, 0)),
            pl.BlockSpec((_BLK, 1), lambda j: (j, 0)),
        ],
        out_specs=[
            pl.BlockSpec((_BLK, WA), lambda j: (j, 0)),
            pl.BlockSpec((_BLK, WB), lambda j: (j, 0)),
        ],
        out_shape=[
            jax.ShapeDtypeStruct((_NROW, WA), jnp.float32),
            jax.ShapeDtypeStruct((_NROW, WB), jnp.float32),
        ],
    )(s2_partials, x2, dinv, rs)


def _tcw_body(wg1, wg2, bg1, w12, rvec):
    w12[...] = jnp.dot(wg1[...], wg2[...],
                       preferred_element_type=jnp.float32)
    rvec[...] = jnp.dot(bg1[...], wg2[...],
                        preferred_element_type=jnp.float32)


def _tc_w(Wg1, Wg2, bg1):
    return pl.pallas_call(
        _tcw_body,
        out_shape=[
            jax.ShapeDtypeStruct((NINP, NINP), jnp.float32),
            jax.ShapeDtypeStruct((1, NINP), jnp.float32),
        ],
    )(Wg1, Wg2, bg1.reshape(1, 2 * NINP))


def _layer_norm(x, g, b):
    m = jnp.mean(x, axis=-1, keepdims=True)
    v = jnp.mean((x - m) ** 2, axis=-1, keepdims=True)
    return (x - m) * lax.rsqrt(v + 1e-5) * g + b


def _tcd_body(raw_ref, rawb_ref, ts_ref, seq_ref, tem_ref, pos_ref, w12_ref, rvec_ref,
              bg2_ref, wq_ref, bq_ref, wk_ref, bk_ref, wv_ref, bv_ref,
              wo_ref, bo_ref, l1g_ref, l1b_ref, l2g_ref, l2b_ref,
              wf1_ref, bf1_ref, wf2_ref, bf2_ref, out_ref):
    raw = raw_ref[...][:, 0]
    rawb = rawb_ref[...][:, 0]
    w12 = w12_ref[...]
    dyu = (jnp.dot(raw.reshape(TSTEPS * L, NINP), w12,
                   preferred_element_type=jnp.float32).reshape(TSTEPS, L, NINP)
           + rawb[..., 0][:, :, None] * rvec_ref[...][None]
           + bg2_ref[...][None])
    ts = ts_ref[0, 0]
    onehot = (ts[:, None] == lax.broadcasted_iota(jnp.int32, (L, TSTEPS), 1))
    tem = jnp.dot(onehot.astype(jnp.float32), tem_ref[...],
                  preferred_element_type=jnp.float32)
    temperature = 128 ** 0.5 + 1e-06
    affine = jnp.sum(tem[None] * dyu, axis=-1) / temperature
    affine = affine - jnp.max(affine, axis=1, keepdims=True)
    ea = jnp.exp(affine)
    alpha = ea / jnp.sum(ea, axis=1, keepdims=True)
    dyemb = jnp.sum(alpha[:, :, None] * dyu, axis=0)
    x = jnp.concatenate([dyemb, pos_ref[...]], axis=-1)

    q = jnp.dot(x, wq_ref[...], preferred_element_type=jnp.float32) + bq_ref[...]
    k = jnp.dot(x, wk_ref[...], preferred_element_type=jnp.float32) + bk_ref[...]
    v = jnp.dot(x, wv_ref[...], preferred_element_type=jnp.float32) + bv_ref[...]
    kmask = (seq_ref[0, 0] == PAD)[None, :]
    heads = []
    scale = 1.0 / (float(HD) ** 0.5)
    for h in range(NHEADS):
        sl = slice(h * HD, (h + 1) * HD)
        qh, kh, vh = q[:, sl], k[:, sl], v[:, sl]
        sc = lax.dot_general(qh, kh, (((1,), (1,)), ((), ())),
                             preferred_element_type=jnp.float32) * scale
        sc = jnp.where(kmask, -1e9, sc)
        sc = sc - jnp.max(sc, axis=-1, keepdims=True)
        es = jnp.exp(sc)
        attn = es / jnp.sum(es, axis=-1, keepdims=True)
        heads.append(jnp.dot(attn, vh, preferred_element_type=jnp.float32))
    o = jnp.concatenate(heads, axis=-1)
    hh = _layer_norm(x + jnp.dot(o, wo_ref[...],
                                 preferred_element_type=jnp.float32)
                     + bo_ref[...], l1g_ref[...], l1b_ref[...])
    ff = jnp.dot(jnp.maximum(jnp.dot(hh, wf1_ref[...],
                                     preferred_element_type=jnp.float32)
                             + bf1_ref[...], 0.0), wf2_ref[...],
                 preferred_element_type=jnp.float32) + bf2_ref[...]
    out_ref[...] = _layer_norm(hh + ff, l2g_ref[...], l2b_ref[...])[None]


def _tc_head(dyu_raw, dyu_rawb, dyemb_ts, seq, time_emb, pos200, w12, rvec,
             bg2, Wq, bq, Wk, bk, Wv, bv, Wo, bo, ln1g, ln1b, ln2g, ln2b,
             Wf1, bf1, Wf2, bf2):
    row = lambda a: a.reshape(1, -1)

    def full(a):
        nd = a.ndim
        return pl.BlockSpec(a.shape, lambda b, _nd=nd: (0,) * _nd)
    ins = [dyu_raw.reshape(TSTEPS, BATCH, L, WA),
           dyu_rawb.reshape(TSTEPS, BATCH, L, WB)]
    specs = [pl.BlockSpec((TSTEPS, 1, L, WA), lambda b: (0, b, 0, 0)),
             pl.BlockSpec((TSTEPS, 1, L, WB), lambda b: (0, b, 0, 0))]
    for a in (dyemb_ts, seq):
        ins.append(a.reshape(BATCH, 1, L))
        specs.append(pl.BlockSpec((1, 1, L), lambda b: (b, 0, 0)))
    for a in (time_emb, pos200, w12, rvec, row(bg2), Wq, row(bq), Wk, row(bk),
              Wv, row(bv), Wo, row(bo), row(ln1g), row(ln1b), row(ln2g),
              row(ln2b), Wf1, row(bf1), Wf2, row(bf2)):
        ins.append(a)
        specs.append(full(a))
    return pl.pallas_call(
        _tcd_body,
        grid=(BATCH,),
        in_specs=specs,
        out_specs=pl.BlockSpec((1, L, D), lambda b: (b, 0, 0)),
        out_shape=jax.ShapeDtypeStruct((BATCH, L, D), jnp.float32),
    )(*ins)


def _tce_body(att_ref, wout_ref, bout_ref, seq_ref, out_ref):
    mm = (jnp.dot(att_ref[0], wout_ref[...],
                  preferred_element_type=jnp.float32) + bout_ref[...])
    viota = lax.broadcasted_iota(jnp.int32, (L, NTOKEN), 1)
    hit = (seq_ref[0, 0][:, None] == viota).astype(jnp.int32)
    sh = 1
    while sh < L:
        z = jnp.zeros((sh, NTOKEN), jnp.int32)
        hit = hit | jnp.concatenate([z, hit[:L - sh]], axis=0)
        sh *= 2
    masked = (hit > 0) | (viota == 0)
    out_ref[...] = jnp.where(masked, -jnp.inf, mm)


def _tc_proj(att_out, Wout, bout, seq):
    return pl.pallas_call(
        _tce_body,
        grid=(BATCH,),
        in_specs=[
            pl.BlockSpec((1, L, D), lambda b: (b, 0, 0)),
            pl.BlockSpec((D, NTOKEN), lambda b: (0, 0)),
            pl.BlockSpec((1, NTOKEN), lambda b: (0, 0)),
            pl.BlockSpec((1, 1, L), lambda b: (b, 0, 0)),
        ],
        out_specs=pl.BlockSpec((L, NTOKEN), lambda b: (b, 0)),
        out_shape=jax.ShapeDtypeStruct((BATCH * L, NTOKEN), jnp.float32),
    )(att_out, Wout, bout.reshape(1, NTOKEN), seq.reshape(BATCH, 1, L))


def _dyemb_ts(input_timestamp):
    ts = input_timestamp[:, :-1]
    Bz, Ls = ts.shape
    pad = (-Ls) % 5
    padded = jnp.pad(ts, ((0, 0), (0, pad)))
    nb = (Ls + pad) // 5
    blocks = padded.reshape(Bz, nb, 5)
    la = blocks.max(axis=(0, 2))
    active = jnp.cumprod((la >= 1).astype(jnp.int32)) > 0
    res_index = jnp.minimum(la, TSTEPS - 1)
    vals = jnp.where(active, res_index, 0).astype(jnp.int32)
    dy = jnp.broadcast_to(vals[None, :, None], (Bz, nb, 5)).reshape(Bz, nb * 5)
    return dy[:, :Ls]


def kernel(input, input_timestamp, diffusion_edge_index, emb, Wg1, bg1, Wg2,
           bg2, pos_emb, time_emb, Wq, bq, Wk, bk, Wv, bv, Wo, bo, ln1g, ln1b,
           ln2g, ln2b, Wf1, bf1, Wf2, bf2, Wout, bout):
    seq = input[:, :-1]
    dyemb_ts = _dyemb_ts(input_timestamp)
    src_flat = diffusion_edge_index[:, 0, :].reshape(-1)
    dst_flat = diffusion_edge_index[:, 1, :].reshape(-1)
    embp = jnp.pad(emb, ((0, NTP - NTOKEN), (0, 0)))

    deg_partials = _sc_degree(dst_flat)
    dinv, x1, x1b = _tc_prep1(deg_partials, embp)
    s1p, s1pb = _sc_hop(x1, src_flat, dst_flat, xb_tables=x1b)
    x2, rs = _tc_prep2(s1p, s1pb, x1, x1b, dinv)
    s2p = _sc_hop(x2, src_flat, dst_flat)
    g, gb = _tc_prep3(s2p, x2, dinv, rs)

    qidx = (jnp.arange(TSTEPS, dtype=jnp.int32)[:, None] * NTP
            + seq.reshape(-1)[None, :]).reshape(-1)
    dyu_raw, dyu_rawb = _sc_qgather(g, gb, qidx)

    w12, rvec = _tc_w(Wg1, Wg2, bg1)
    pos200 = pos_emb[:L]
    att_out = _tc_head(dyu_raw, dyu_rawb, dyemb_ts, seq, time_emb, pos200,
                       w12, rvec, bg2, Wq, bq, Wk, bk, Wv, bv, Wo, bo,
                       ln1g, ln1b, ln2g, ln2b, Wf1, bf1, Wf2, bf2)
    return _tc_proj(att_out, Wout, bout, seq)


# speedup vs baseline: 15.2935x; 1.1740x over previous
import functools

import jax
import jax.numpy as jnp
from jax import lax
from jax.experimental import pallas as pl
from jax.experimental.pallas import tpu as pltpu
from jax.experimental.pallas import tpu_sc as plsc

PAD = 0
NTOKEN = 10000
NINP = 128
POS_DIM = 8
TSTEPS = 8
NHEADS = 8
BATCH = 8
SEQ = 201
NEDGES = 320000

NTP = 10240
WA = 128
WB = 16
NC, NS, NW = 2, 16, 32
EPT = NEDGES // NW
CH = 64
L = SEQ - 1
D = NINP + POS_DIM
HD = D // NHEADS
NQ = BATCH * L
QPT = NQ * TSTEPS // NW
RPT = NTP // NS

_mesh = functools.partial(plsc.VectorSubcoreMesh,
                          core_axis_name="c", subcore_axis_name="s")


def _deg_body(dst_hbm, out_hbm, dstr, hist):
    cid = lax.axis_index("c")
    sid = lax.axis_index("s")
    wid = cid * NS + sid
    zv = jnp.zeros((16,), jnp.float32)
    ones = jnp.ones((16,), jnp.float32)

    def zero_step(i, _):
        hist[i, :] = zv
        return 0
    lax.fori_loop(0, TSTEPS * NTP // 16, zero_step, 0)

    def t_step(t, _):
        pltpu.sync_copy(dst_hbm.at[pl.ds(t * NEDGES + wid * EPT, EPT)], dstr)

        def e_step(i, _):
            idx = dstr[pl.ds(i * 16, 16)] + t * NTP
            plsc.addupdate_scatter(hist, [idx >> 4, idx & 15], ones)
            return 0
        lax.fori_loop(0, EPT // 16, e_step, 0)
        return 0
    lax.fori_loop(0, TSTEPS, t_step, 0)
    pltpu.sync_copy(hist, out_hbm.at[wid])


def _sc_degree(dst_flat):
    k = pl.kernel(
        _deg_body,
        out_type=jax.ShapeDtypeStruct((NW, TSTEPS * NTP // 16, 16), jnp.float32),
        mesh=_mesh(),
        scratch_types=[
            pltpu.VMEM((EPT,), jnp.int32),
            pltpu.VMEM((TSTEPS * NTP // 16, 16), jnp.float32),
        ],
        compiler_params=pltpu.CompilerParams(needs_layout_passes=False, use_tc_tiling_on_sc=False),
    )
    return k(dst_flat).reshape(NW, TSTEPS * NTP)


HB = EPT // 2
NCH2 = (HB + CH - 1) // CH
SPAD = NCH2 * CH


def _hop_body(has_aux, *args):
    if has_aux:
        (x_hbm, xb_hbm, src_hbm, dst_hbm, out_hbm, outb_hbm, srcr, dstr, dsts,
         rows0, rows1, rowsb0, rowsb1, acc, accb, sem0, sem1, semb0, semb1) = args
    else:
        (x_hbm, src_hbm, dst_hbm, out_hbm, srcr, dstr, dsts,
         rows0, rows1, acc, sem0, sem1) = args
        rowsb0 = rowsb1 = semb0 = semb1 = None
    cid = lax.axis_index("c")
    sid = lax.axis_index("s")
    wid = cid * NS + sid
    zv = jnp.zeros((16,), jnp.float32)
    iota = jnp.arange(16, dtype=jnp.int32)

    def start_gather(j, rows, sem, rowsb, semb):
        pltpu.async_copy(x_hbm.at[srcr.at[pl.ds(j * CH, CH)]], rows, sem)
        if has_aux:
            pltpu.async_copy(xb_hbm.at[srcr.at[pl.ds(j * CH, CH)]], rowsb, semb)

    def t_step(t, _):
        def z_step(i, _):
            j = i // (WA // 16)
            k = i % (WA // 16)
            rows0[j, pl.ds(k * 16, 16)] = zv
            return 0
        lax.fori_loop(0, CH * (WA // 16), z_step, 0)

        def clr(m, _):
            pltpu.sync_copy(rows0, acc.at[pl.ds(sid * RPT + m * CH, CH)])
            return 0
        lax.fori_loop(0, RPT // CH, clr, 0)
        if has_aux:
            def zb_step(i, _):
                rowsb0[i, :] = zv
                return 0
            lax.fori_loop(0, CH, zb_step, 0)

            def clrb(m, _):
                pltpu.sync_copy(rowsb0, accb.at[pl.ds(sid * RPT + m * CH, CH)])
                return 0
            lax.fori_loop(0, RPT // CH, clrb, 0)

        gpad = t * NTP + NTOKEN + sid * 8
        spad = NTOKEN + sid * 8 + cid

        def batch(b2, _):
            ebase = t * NEDGES + wid * EPT + b2 * HB
            pltpu.sync_copy(src_hbm.at[pl.ds(ebase, HB)],
                            srcr.at[pl.ds(0, HB)])
            pltpu.sync_copy(dst_hbm.at[pl.ds(ebase, HB)],
                            dstr.at[pl.ds(0, HB)])

            def stage(i, _):
                j = i // (CH // 16)
                k = i % (CH // 16)
                pos = i * 16
                valid = (pos + iota) < HB
                sv = srcr[pl.ds(pos, 16)]
                dv = dstr[pl.ds(pos, 16)]
                srcr[pl.ds(pos, 16)] = jnp.where(valid, sv + t * NTP, gpad)
                dsts[j, pl.ds(k * 16, 16)] = jnp.where(valid, dv, spad)
                return 0
            lax.fori_loop(0, NCH2 * (CH // 16), stage, 0)

            start_gather(0, rows0, sem0, rowsb0, semb0)

            def chunk(j, _):
                def phase(rows, sem, rowsb, semb, orows, osem, orowsb, osemb):
                    pltpu.make_async_copy(
                        x_hbm.at[srcr.at[pl.ds(j * CH, CH)]], rows, sem).wait()
                    if has_aux:
                        pltpu.make_async_copy(
                            xb_hbm.at[srcr.at[pl.ds(j * CH, CH)]],
                            rowsb, semb).wait()
                    lax.cond(j + 1 < NCH2,
                             lambda: start_gather(
                                 jnp.minimum(j + 1, NCH2 - 1),
                                 orows, osem, orowsb, osemb),
                             lambda: None)
                    pltpu.sync_copy(rows, acc.at[dsts.at[j]], add=True)
                    if has_aux:
                        pltpu.sync_copy(rowsb, accb.at[dsts.at[j]], add=True)
                lax.cond(j % 2 == 0,
                         lambda: phase(rows0, sem0, rowsb0, semb0,
                                       rows1, sem1, rowsb1, semb1),
                         lambda: phase(rows1, sem1, rowsb1, semb1,
                                       rows0, sem0, rowsb0, semb0))
                return 0
            lax.fori_loop(0, NCH2, chunk, 0)
            return 0
        lax.fori_loop(0, 2, batch, 0)

        plsc.subcore_barrier()

        def wout(m, _):
            r0 = sid * RPT + m * CH
            pltpu.sync_copy(acc.at[pl.ds(r0, CH)],
                            out_hbm.at[cid, t, pl.ds(r0, CH), :])
            return 0
        lax.fori_loop(0, RPT // CH, wout, 0)
        if has_aux:
            def woutb(m, _):
                r0 = sid * RPT + m * CH
                pltpu.sync_copy(accb.at[pl.ds(r0, CH)],
                                outb_hbm.at[cid, t, pl.ds(r0, CH), :])
                return 0
            lax.fori_loop(0, RPT // CH, woutb, 0)

        plsc.subcore_barrier()
        return 0
    lax.fori_loop(0, TSTEPS, t_step, 0)


def _sc_hop(x_tables, src_flat, dst_flat, xb_tables=None):
    has_aux = xb_tables is not None
    out_type = [jax.ShapeDtypeStruct((NC, TSTEPS, NTP, WA), jnp.float32)]
    scratch = [
        pltpu.VMEM((SPAD,), jnp.int32),
        pltpu.VMEM((SPAD,), jnp.int32),
        pltpu.VMEM((NCH2, CH), jnp.int32),
        pltpu.VMEM((CH, WA), jnp.float32),
        pltpu.VMEM((CH, WA), jnp.float32),
    ]
    if has_aux:
        out_type.append(jax.ShapeDtypeStruct((NC, TSTEPS, NTP, WB), jnp.float32))
        scratch.append(pltpu.VMEM((CH, WB), jnp.float32))
        scratch.append(pltpu.VMEM((CH, WB), jnp.float32))
    scratch.append(pltpu.VMEM_SHARED((NTP, WA), jnp.float32))
    if has_aux:
        scratch.append(pltpu.VMEM_SHARED((NTP, WB), jnp.float32))
    scratch.append(pltpu.SemaphoreType.DMA)
    scratch.append(pltpu.SemaphoreType.DMA)
    if has_aux:
        scratch.append(pltpu.SemaphoreType.DMA)
        scratch.append(pltpu.SemaphoreType.DMA)
    k = pl.kernel(
        functools.partial(_hop_body, has_aux),
        out_type=out_type if has_aux else out_type[0],
        name=f"sc_hop_aux{int(has_aux)}",
        mesh=_mesh(),
        scratch_types=scratch,
        compiler_params=pltpu.CompilerParams(needs_layout_passes=False,
                                             use_tc_tiling_on_sc=False),
    )
    if has_aux:
        s, sb = k(x_tables, xb_tables, src_flat, dst_flat)
        return (s.reshape(NC, TSTEPS * NTP, WA),
                sb.reshape(NC, TSTEPS * NTP, WB))
    s = k(x_tables, src_flat, dst_flat)
    return s.reshape(NC, TSTEPS * NTP, WA)


def _qgather_body(g_hbm, gb_hbm, qidx_hbm, out_hbm, outb_hbm,
                  qv, rows, rowsb, sem):
    cid = lax.axis_index("c")
    sid = lax.axis_index("s")
    wid = cid * NS + sid
    base = wid * QPT
    pltpu.sync_copy(qidx_hbm.at[pl.ds(base, QPT)], qv)

    def chunk(j, _):
        pltpu.async_copy(g_hbm.at[qv.at[pl.ds(j * 80, 80)]], rows, sem).wait()
        pltpu.sync_copy(rows, out_hbm.at[pl.ds(base + j * 80, 80), :])
        pltpu.async_copy(gb_hbm.at[qv.at[pl.ds(j * 80, 80)]], rowsb, sem).wait()
        pltpu.sync_copy(rowsb, outb_hbm.at[pl.ds(base + j * 80, 80), :])
        return 0
    lax.fori_loop(0, QPT // 80, chunk, 0)


def _sc_qgather(g_tables, gb_tables, qidx):
    k = pl.kernel(
        _qgather_body,
        out_type=[jax.ShapeDtypeStruct((TSTEPS * NQ, WA), jnp.float32),
                  jax.ShapeDtypeStruct((TSTEPS * NQ, WB), jnp.float32)],
        mesh=_mesh(),
        scratch_types=[
            pltpu.VMEM((QPT,), jnp.int32),
            pltpu.VMEM((80, WA), jnp.float32),
            pltpu.VMEM((80, WB), jnp.float32),
            pltpu.SemaphoreType.DMA,
        ],
        compiler_params=pltpu.CompilerParams(needs_layout_passes=False, use_tc_tiling_on_sc=False),
    )
    return k(g_tables, gb_tables, qidx)


_BLK = 640
_NROW = TSTEPS * NTP
_NBLK = _NROW // _BLK


def _tca_body(degp, emb, dinv, x1, x1b):
    deg = jnp.sum(degp[...], axis=0) + 1.0
    dv = lax.rsqrt(deg)[:, None]
    dinv[...] = dv
    x1[...] = dv * emb[...]
    x1b[...] = jnp.concatenate(
        [dv, jnp.zeros((_BLK, WB - 1), jnp.float32)], axis=-1)


def _tc_prep1(deg_partials, embp8):
    return pl.pallas_call(
        _tca_body,
        grid=(_NBLK,),
        in_specs=[
            pl.BlockSpec((NW, _BLK), lambda j: (0, j)),
            pl.BlockSpec((_BLK, NINP), lambda j: (j % (NTP // _BLK), 0)),
        ],
        out_specs=[
            pl.BlockSpec((_BLK, 1), lambda j: (j, 0)),
            pl.BlockSpec((_BLK, WA), lambda j: (j, 0)),
            pl.BlockSpec((_BLK, WB), lambda j: (j, 0)),
        ],
        out_shape=[
            jax.ShapeDtypeStruct((_NROW, 1), jnp.float32),
            jax.ShapeDtypeStruct((_NROW, WA), jnp.float32),
            jax.ShapeDtypeStruct((_NROW, WB), jnp.float32),
        ],
    )(deg_partials, embp8)


def _tcb_body(s1p, s1pb, x1, x1b, dinv, x2, rs):
    s = s1p[0] + s1p[1] + x1[...]
    p = dinv[...] * s
    x2[...] = dinv[...] * p
    sb = s1pb[0] + s1pb[1] + x1b[...]
    rs[...] = dinv[...] * sb[:, 0:1]


def _tc_prep2(s1_partials, s1b_partials, x1, x1b, dinv):
    return pl.pallas_call(
        _tcb_body,
        grid=(_NBLK,),
        in_specs=[
            pl.BlockSpec((NC, _BLK, WA), lambda j: (0, j, 0)),
            pl.BlockSpec((NC, _BLK, WB), lambda j: (0, j, 0)),
            pl.BlockSpec((_BLK, WA), lambda j: (j, 0)),
            pl.BlockSpec((_BLK, WB), lambda j: (j, 0)),
            pl.BlockSpec((_BLK, 1), lambda j: (j, 0)),
        ],
        out_specs=[
            pl.BlockSpec((_BLK, WA), lambda j: (j, 0)),
            pl.BlockSpec((_BLK, 1), lambda j: (j, 0)),
        ],
        out_shape=[
            jax.ShapeDtypeStruct((_NROW, WA), jnp.float32),
            jax.ShapeDtypeStruct((_NROW, 1), jnp.float32),
        ],
    )(s1_partials, s1b_partials, x1, x1b, dinv)


def _tcc_body(s2p, x2, dinv, rs, g, gb):
    s = s2p[0] + s2p[1] + x2[...]
    g[...] = dinv[...] * s
    gb[...] = jnp.concatenate(
        [rs[...], jnp.zeros((_BLK, WB - 1), jnp.float32)], axis=-1)


def _tc_prep3(s2_partials, x2, dinv, rs):
    return pl.pallas_call(
        _tcc_body,
        grid=(_NBLK,),
        in_specs=[
            pl.BlockSpec((NC, _BLK, WA), lambda j: (0, j, 0)),
            pl.BlockSpec((_BLK, WA), lambda j: (j, 0)),
            pl.BlockSpec((_BLK, 1), lambda j: (j, 0)),
            pl.BlockSpec((_BLK, 1), lambda j: (j, 0)),
        ],
        out_specs=[
            pl.BlockSpec((_BLK, WA), lambda j: (j, 0)),
            pl.BlockSpec((_BLK, WB), lambda j: (j, 0)),
        ],
        out_shape=[
            jax.ShapeDtypeStruct((_NROW, WA), jnp.float32),
            jax.ShapeDtypeStruct((_NROW, WB), jnp.float32),
        ],
    )(s2_partials, x2, dinv, rs)


def _tcw_body(wg1, wg2, bg1, w12, rvec):
    w12[...] = jnp.dot(wg1[...], wg2[...],
                       preferred_element_type=jnp.float32)
    rvec[...] = jnp.dot(bg1[...], wg2[...],
                        preferred_element_type=jnp.float32)


def _tc_w(Wg1, Wg2, bg1):
    return pl.pallas_call(
        _tcw_body,
        out_shape=[
            jax.ShapeDtypeStruct((NINP, NINP), jnp.float32),
            jax.ShapeDtypeStruct((1, NINP), jnp.float32),
        ],
    )(Wg1, Wg2, bg1.reshape(1, 2 * NINP))


def _layer_norm(x, g, b):
    m = jnp.mean(x, axis=-1, keepdims=True)
    v = jnp.mean((x - m) ** 2, axis=-1, keepdims=True)
    return (x - m) * lax.rsqrt(v + 1e-5) * g + b


def _tcd_body(raw_ref, rawb_ref, ts_ref, seq_ref, tem_ref, pos_ref, w12_ref, rvec_ref,
              bg2_ref, wq_ref, bq_ref, wk_ref, bk_ref, wv_ref, bv_ref,
              wo_ref, bo_ref, l1g_ref, l1b_ref, l2g_ref, l2b_ref,
              wf1_ref, bf1_ref, wf2_ref, bf2_ref, out_ref):
    raw = raw_ref[...][:, 0]
    rawb = rawb_ref[...][:, 0]
    w12 = w12_ref[...]
    dyu = (jnp.dot(raw.reshape(TSTEPS * L, NINP), w12,
                   preferred_element_type=jnp.float32).reshape(TSTEPS, L, NINP)
           + rawb[..., 0][:, :, None] * rvec_ref[...][None]
           + bg2_ref[...][None])
    ts = ts_ref[0, 0]
    onehot = (ts[:, None] == lax.broadcasted_iota(jnp.int32, (L, TSTEPS), 1))
    tem = jnp.dot(onehot.astype(jnp.float32), tem_ref[...],
                  preferred_element_type=jnp.float32)
    temperature = 128 ** 0.5 + 1e-06
    affine = jnp.sum(tem[None] * dyu, axis=-1) / temperature
    affine = affine - jnp.max(affine, axis=1, keepdims=True)
    ea = jnp.exp(affine)
    alpha = ea / jnp.sum(ea, axis=1, keepdims=True)
    dyemb = jnp.sum(alpha[:, :, None] * dyu, axis=0)
    x = jnp.concatenate([dyemb, pos_ref[...]], axis=-1)

    q = jnp.dot(x, wq_ref[...], preferred_element_type=jnp.float32) + bq_ref[...]
    k = jnp.dot(x, wk_ref[...], preferred_element_type=jnp.float32) + bk_ref[...]
    v = jnp.dot(x, wv_ref[...], preferred_element_type=jnp.float32) + bv_ref[...]
    kmask = (seq_ref[0, 0] == PAD)[None, :]
    heads = []
    scale = 1.0 / (float(HD) ** 0.5)
    for h in range(NHEADS):
        sl = slice(h * HD, (h + 1) * HD)
        qh, kh, vh = q[:, sl], k[:, sl], v[:, sl]
        sc = lax.dot_general(qh, kh, (((1,), (1,)), ((), ())),
                             preferred_element_type=jnp.float32) * scale
        sc = jnp.where(kmask, -1e9, sc)
        sc = sc - jnp.max(sc, axis=-1, keepdims=True)
        es = jnp.exp(sc)
        attn = es / jnp.sum(es, axis=-1, keepdims=True)
        heads.append(jnp.dot(attn, vh, preferred_element_type=jnp.float32))
    o = jnp.concatenate(heads, axis=-1)
    hh = _layer_norm(x + jnp.dot(o, wo_ref[...],
                                 preferred_element_type=jnp.float32)
                     + bo_ref[...], l1g_ref[...], l1b_ref[...])
    ff = jnp.dot(jnp.maximum(jnp.dot(hh, wf1_ref[...],
                                     preferred_element_type=jnp.float32)
                             + bf1_ref[...], 0.0), wf2_ref[...],
                 preferred_element_type=jnp.float32) + bf2_ref[...]
    out_ref[...] = _layer_norm(hh + ff, l2g_ref[...], l2b_ref[...])[None]


def _tc_head(dyu_raw, dyu_rawb, dyemb_ts, seq, time_emb, pos200, w12, rvec,
             bg2, Wq, bq, Wk, bk, Wv, bv, Wo, bo, ln1g, ln1b, ln2g, ln2b,
             Wf1, bf1, Wf2, bf2):
    row = lambda a: a.reshape(1, -1)

    def full(a):
        nd = a.ndim
        return pl.BlockSpec(a.shape, lambda b, _nd=nd: (0,) * _nd)
    ins = [dyu_raw.reshape(TSTEPS, BATCH, L, WA),
           dyu_rawb.reshape(TSTEPS, BATCH, L, WB)]
    specs = [pl.BlockSpec((TSTEPS, 1, L, WA), lambda b: (0, b, 0, 0)),
             pl.BlockSpec((TSTEPS, 1, L, WB), lambda b: (0, b, 0, 0))]
    for a in (dyemb_ts, seq):
        ins.append(a.reshape(BATCH, 1, L))
        specs.append(pl.BlockSpec((1, 1, L), lambda b: (b, 0, 0)))
    for a in (time_emb, pos200, w12, rvec, row(bg2), Wq, row(bq), Wk, row(bk),
              Wv, row(bv), Wo, row(bo), row(ln1g), row(ln1b), row(ln2g),
              row(ln2b), Wf1, row(bf1), Wf2, row(bf2)):
        ins.append(a)
        specs.append(full(a))
    return pl.pallas_call(
        _tcd_body,
        grid=(BATCH,),
        in_specs=specs,
        out_specs=pl.BlockSpec((1, L, D), lambda b: (b, 0, 0)),
        out_shape=jax.ShapeDtypeStruct((BATCH, L, D), jnp.float32),
    )(*ins)


def _tce_body(att_ref, wout_ref, bout_ref, seq_ref, out_ref):
    mm = (jnp.dot(att_ref[0], wout_ref[...],
                  preferred_element_type=jnp.float32) + bout_ref[...])
    viota = lax.broadcasted_iota(jnp.int32, (L, NTOKEN), 1)
    hit = (seq_ref[0, 0][:, None] == viota).astype(jnp.int32)
    sh = 1
    while sh < L:
        z = jnp.zeros((sh, NTOKEN), jnp.int32)
        hit = hit | jnp.concatenate([z, hit[:L - sh]], axis=0)
        sh *= 2
    masked = (hit > 0) | (viota == 0)
    out_ref[...] = jnp.where(masked, -jnp.inf, mm)


def _tc_proj(att_out, Wout, bout, seq):
    return pl.pallas_call(
        _tce_body,
        grid=(BATCH,),
        in_specs=[
            pl.BlockSpec((1, L, D), lambda b: (b, 0, 0)),
            pl.BlockSpec((D, NTOKEN), lambda b: (0, 0)),
            pl.BlockSpec((1, NTOKEN), lambda b: (0, 0)),
            pl.BlockSpec((1, 1, L), lambda b: (b, 0, 0)),
        ],
        out_specs=pl.BlockSpec((L, NTOKEN), lambda b: (b, 0)),
        out_shape=jax.ShapeDtypeStruct((BATCH * L, NTOKEN), jnp.float32),
    )(att_out, Wout, bout.reshape(1, NTOKEN), seq.reshape(BATCH, 1, L))


def _dyemb_ts(input_timestamp):
    ts = input_timestamp[:, :-1]
    Bz, Ls = ts.shape
    pad = (-Ls) % 5
    padded = jnp.pad(ts, ((0, 0), (0, pad)))
    nb = (Ls + pad) // 5
    blocks = padded.reshape(Bz, nb, 5)
    la = blocks.max(axis=(0, 2))
    active = jnp.cumprod((la >= 1).astype(jnp.int32)) > 0
    res_index = jnp.minimum(la, TSTEPS - 1)
    vals = jnp.where(active, res_index, 0).astype(jnp.int32)
    dy = jnp.broadcast_to(vals[None, :, None], (Bz, nb, 5)).reshape(Bz, nb * 5)
    return dy[:, :Ls]


def kernel(input, input_timestamp, diffusion_edge_index, emb, Wg1, bg1, Wg2,
           bg2, pos_emb, time_emb, Wq, bq, Wk, bk, Wv, bv, Wo, bo, ln1g, ln1b,
           ln2g, ln2b, Wf1, bf1, Wf2, bf2, Wout, bout):
    seq = input[:, :-1]
    dyemb_ts = _dyemb_ts(input_timestamp)
    src_flat = diffusion_edge_index[:, 0, :].reshape(-1)
    dst_flat = diffusion_edge_index[:, 1, :].reshape(-1)
    embp = jnp.pad(emb, ((0, NTP - NTOKEN), (0, 0)))

    deg_partials = _sc_degree(dst_flat)
    dinv, x1, x1b = _tc_prep1(deg_partials, embp)
    s1p, s1pb = _sc_hop(x1, src_flat, dst_flat, xb_tables=x1b)
    x2, rs = _tc_prep2(s1p, s1pb, x1, x1b, dinv)
    s2p = _sc_hop(x2, src_flat, dst_flat)
    g, gb = _tc_prep3(s2p, x2, dinv, rs)

    qidx = (jnp.arange(TSTEPS, dtype=jnp.int32)[:, None] * NTP
            + seq.reshape(-1)[None, :]).reshape(-1)
    dyu_raw, dyu_rawb = _sc_qgather(g, gb, qidx)

    w12, rvec = _tc_w(Wg1, Wg2, bg1)
    pos200 = pos_emb[:L]
    att_out = _tc_head(dyu_raw, dyu_rawb, dyemb_ts, seq, time_emb, pos200,
                       w12, rvec, bg2, Wq, bq, Wk, bk, Wv, bv, Wo, bo,
                       ln1g, ln1b, ln2g, ln2b, Wf1, bf1, Wf2, bf2)
    return _tc_proj(att_out, Wout, bout, seq)


# speedup vs baseline: 20.9754x; 1.3715x over previous
import functools

import jax
import jax.numpy as jnp
from jax import lax
from jax.experimental import pallas as pl
from jax.experimental.pallas import tpu as pltpu
from jax.experimental.pallas import tpu_sc as plsc

PAD = 0
NTOKEN = 10000
NINP = 128
POS_DIM = 8
TSTEPS = 8
NHEADS = 8
BATCH = 8
SEQ = 201
NEDGES = 320000

NTP = 10240
WA = 128
WB = 16
NC, NS, NW = 2, 16, 32
EPT = NEDGES // NW
CH = 64
L = SEQ - 1
D = NINP + POS_DIM
HD = D // NHEADS
NQ = BATCH * L
QPT = NQ * TSTEPS // NW
RPT = NTP // NS

_mesh = functools.partial(plsc.VectorSubcoreMesh,
                          core_axis_name="c", subcore_axis_name="s")


def _deg_body(dst_hbm, out_hbm, dstr, hist):
    cid = lax.axis_index("c")
    sid = lax.axis_index("s")
    wid = cid * NS + sid
    zv = jnp.zeros((16,), jnp.float32)
    ones = jnp.ones((16,), jnp.float32)

    def zero_step(i, _):
        hist[i, :] = zv
        return 0
    lax.fori_loop(0, TSTEPS * NTP // 16, zero_step, 0)

    def t_step(t, _):
        pltpu.sync_copy(dst_hbm.at[pl.ds(t * NEDGES + wid * EPT, EPT)], dstr)

        def e_step(i, _):
            idx = dstr[pl.ds(i * 16, 16)] + t * NTP
            plsc.addupdate_scatter(hist, [idx >> 4, idx & 15], ones)
            return 0
        lax.fori_loop(0, EPT // 16, e_step, 0)
        return 0
    lax.fori_loop(0, TSTEPS, t_step, 0)
    pltpu.sync_copy(hist, out_hbm.at[wid])


def _sc_degree(dst_flat):
    k = pl.kernel(
        _deg_body,
        out_type=jax.ShapeDtypeStruct((NW, TSTEPS * NTP // 16, 16), jnp.float32),
        mesh=_mesh(),
        scratch_types=[
            pltpu.VMEM((EPT,), jnp.int32),
            pltpu.VMEM((TSTEPS * NTP // 16, 16), jnp.float32),
        ],
        compiler_params=pltpu.CompilerParams(needs_layout_passes=False, use_tc_tiling_on_sc=False),
    )
    return k(dst_flat).reshape(NW, TSTEPS * NTP)


HB = EPT // 2
NCH2 = (HB + CH - 1) // CH
SPAD = NCH2 * CH


def _hop_body(has_aux, filt, *args):
    qflag_hbm = qflag_v = None
    rowsb0 = rowsb1 = semb0 = semb1 = None
    if has_aux:
        (x_hbm, xb_hbm, src_hbm, dst_hbm, out_hbm, outb_hbm, srcr, dstr, dsts,
         rows0, rows1, rowsb0, rowsb1, acc, accb, sem0, sem1, semb0, semb1) = args
    elif filt:
        (x_hbm, src_hbm, dst_hbm, qflag_hbm, out_hbm, srcr, dstr, dsts,
         rows0, rows1, qflag_v, acc, sem0, sem1) = args
    else:
        (x_hbm, src_hbm, dst_hbm, out_hbm, srcr, dstr, dsts,
         rows0, rows1, acc, sem0, sem1) = args
    cid = lax.axis_index("c")
    sid = lax.axis_index("s")
    wid = cid * NS + sid
    zv = jnp.zeros((16,), jnp.float32)
    iota = jnp.arange(16, dtype=jnp.int32)
    if filt:
        pltpu.sync_copy(qflag_hbm, qflag_v)

    def start_gather(j, rows, sem, rowsb, semb):
        pltpu.async_copy(x_hbm.at[srcr.at[pl.ds(j * CH, CH)]], rows, sem)
        if has_aux:
            pltpu.async_copy(xb_hbm.at[srcr.at[pl.ds(j * CH, CH)]], rowsb, semb)

    def t_step(t, _):
        def z_step(i, _):
            j = i // (WA // 16)
            k = i % (WA // 16)
            rows0[j, pl.ds(k * 16, 16)] = zv
            return 0
        lax.fori_loop(0, CH * (WA // 16), z_step, 0)

        def clr(m, _):
            pltpu.sync_copy(rows0, acc.at[pl.ds(sid * RPT + m * CH, CH)])
            return 0
        lax.fori_loop(0, RPT // CH, clr, 0)
        if has_aux:
            def zb_step(i, _):
                rowsb0[i, :] = zv
                return 0
            lax.fori_loop(0, CH, zb_step, 0)

            def clrb(m, _):
                pltpu.sync_copy(rowsb0, accb.at[pl.ds(sid * RPT + m * CH, CH)])
                return 0
            lax.fori_loop(0, RPT // CH, clrb, 0)

        gpad = t * NTP + NTOKEN + sid * 8
        spad = NTOKEN + sid * 8 + cid

        def batch(b2, _):
            ebase = t * NEDGES + wid * EPT + b2 * HB
            pltpu.sync_copy(src_hbm.at[pl.ds(ebase, HB)],
                            srcr.at[pl.ds(0, HB)])
            pltpu.sync_copy(dst_hbm.at[pl.ds(ebase, HB)],
                            dstr.at[pl.ds(0, HB)])

            if filt:
                def cstage(i, cur):
                    pos = i * 16
                    valid = (pos + iota) < HB
                    sv = srcr[pl.ds(pos, 16)]
                    dv = dstr[pl.ds(pos, 16)]
                    dv0 = jnp.where(valid, dv, 0)
                    f = plsc.load_gather(qflag_v, [dv0])
                    keep = valid & (f > 0)
                    plsc.store_compressed(srcr.at[pl.ds(cur, 16)],
                                          sv + t * NTP, mask=keep)
                    plsc.store_compressed(dstr.at[pl.ds(cur, 16)],
                                          dv0, mask=keep)
                    return cur + plsc.all_reduce_population_count(keep)[0]
                count = lax.fori_loop(0, (HB + 15) // 16, cstage, 0)
                nch = (count + CH - 1) // CH

                def pfill(i, _):
                    g = (count // 16 + i) * 16
                    m = (g + iota) >= count
                    sv = srcr[pl.ds(g, 16)]
                    dv = dstr[pl.ds(g, 16)]
                    srcr[pl.ds(g, 16)] = jnp.where(m, gpad, sv)
                    dstr[pl.ds(g, 16)] = jnp.where(m, spad, dv)
                    return 0
                lax.fori_loop(0, (CH // 16) + 1, pfill, 0)

                def move(i, _):
                    j = i // (CH // 16)
                    k = i % (CH // 16)
                    dsts[j, pl.ds(k * 16, 16)] = dstr[pl.ds(i * 16, 16)]
                    return 0
                lax.fori_loop(0, nch * (CH // 16), move, 0)
            else:
                nch = NCH2

                def stage(i, _):
                    j = i // (CH // 16)
                    k = i % (CH // 16)
                    pos = i * 16
                    valid = (pos + iota) < HB
                    sv = srcr[pl.ds(pos, 16)]
                    dv = dstr[pl.ds(pos, 16)]
                    srcr[pl.ds(pos, 16)] = jnp.where(valid, sv + t * NTP, gpad)
                    dsts[j, pl.ds(k * 16, 16)] = jnp.where(valid, dv, spad)
                    return 0
                lax.fori_loop(0, NCH2 * (CH // 16), stage, 0)

            lax.cond(nch > 0,
                     lambda: start_gather(0, rows0, sem0, rowsb0, semb0),
                     lambda: None)

            def chunk(j, _):
                def phase(rows, sem, rowsb, semb, orows, osem, orowsb, osemb):
                    pltpu.make_async_copy(
                        x_hbm.at[srcr.at[pl.ds(j * CH, CH)]], rows, sem).wait()
                    if has_aux:
                        pltpu.make_async_copy(
                            xb_hbm.at[srcr.at[pl.ds(j * CH, CH)]],
                            rowsb, semb).wait()
                    lax.cond(j + 1 < nch,
                             lambda: start_gather(
                                 jnp.minimum(j + 1, NCH2 - 1),
                                 orows, osem, orowsb, osemb),
                             lambda: None)
                    pltpu.sync_copy(rows, acc.at[dsts.at[j]], add=True)
                    if has_aux:
                        pltpu.sync_copy(rowsb, accb.at[dsts.at[j]], add=True)
                lax.cond(j % 2 == 0,
                         lambda: phase(rows0, sem0, rowsb0, semb0,
                                       rows1, sem1, rowsb1, semb1),
                         lambda: phase(rows1, sem1, rowsb1, semb1,
                                       rows0, sem0, rowsb0, semb0))
                return 0
            lax.fori_loop(0, nch, chunk, 0)
            return 0
        lax.fori_loop(0, 2, batch, 0)

        plsc.subcore_barrier()

        def wout(m, _):
            r0 = sid * RPT + m * CH
            pltpu.sync_copy(acc.at[pl.ds(r0, CH)],
                            out_hbm.at[cid, t, pl.ds(r0, CH), :])
            return 0
        lax.fori_loop(0, RPT // CH, wout, 0)
        if has_aux:
            def woutb(m, _):
                r0 = sid * RPT + m * CH
                pltpu.sync_copy(accb.at[pl.ds(r0, CH)],
                                outb_hbm.at[cid, t, pl.ds(r0, CH), :])
                return 0
            lax.fori_loop(0, RPT // CH, woutb, 0)

        plsc.subcore_barrier()
        return 0
    lax.fori_loop(0, TSTEPS, t_step, 0)


def _sc_hop(x_tables, src_flat, dst_flat, xb_tables=None, qflag=None):
    has_aux = xb_tables is not None
    filt = qflag is not None
    out_type = [jax.ShapeDtypeStruct((NC, TSTEPS, NTP, WA), jnp.float32)]
    scratch = [
        pltpu.VMEM((SPAD,), jnp.int32),
        pltpu.VMEM((SPAD,), jnp.int32),
        pltpu.VMEM((NCH2, CH), jnp.int32),
        pltpu.VMEM((CH, WA), jnp.float32),
        pltpu.VMEM((CH, WA), jnp.float32),
    ]
    if has_aux:
        out_type.append(jax.ShapeDtypeStruct((NC, TSTEPS, NTP, WB), jnp.float32))
        scratch.append(pltpu.VMEM((CH, WB), jnp.float32))
        scratch.append(pltpu.VMEM((CH, WB), jnp.float32))
    if filt:
        scratch.append(pltpu.VMEM((NTP,), jnp.int32))
    scratch.append(pltpu.VMEM_SHARED((NTP, WA), jnp.float32))
    if has_aux:
        scratch.append(pltpu.VMEM_SHARED((NTP, WB), jnp.float32))
    nsem = 4 if has_aux else 2
    for _ in range(nsem):
        scratch.append(pltpu.SemaphoreType.DMA)
    k = pl.kernel(
        functools.partial(_hop_body, has_aux, filt),
        out_type=out_type if has_aux else out_type[0],
        name=f"sc_hop_aux{int(has_aux)}_f{int(filt)}",
        mesh=_mesh(),
        scratch_types=scratch,
        compiler_params=pltpu.CompilerParams(needs_layout_passes=False,
                                             use_tc_tiling_on_sc=False),
    )
    if has_aux:
        s, sb = k(x_tables, xb_tables, src_flat, dst_flat)
        return (s.reshape(NC, TSTEPS * NTP, WA),
                sb.reshape(NC, TSTEPS * NTP, WB))
    if filt:
        s = k(x_tables, src_flat, dst_flat, qflag)
    else:
        s = k(x_tables, src_flat, dst_flat)
    return s.reshape(NC, TSTEPS * NTP, WA)


def _qgather_body(g_hbm, gb_hbm, qidx_hbm, out_hbm, outb_hbm,
                  qv, rows, rowsb, sem):
    cid = lax.axis_index("c")
    sid = lax.axis_index("s")
    wid = cid * NS + sid
    base = wid * QPT
    pltpu.sync_copy(qidx_hbm.at[pl.ds(base, QPT)], qv)

    def chunk(j, _):
        pltpu.async_copy(g_hbm.at[qv.at[pl.ds(j * 80, 80)]], rows, sem).wait()
        pltpu.sync_copy(rows, out_hbm.at[pl.ds(base + j * 80, 80), :])
        pltpu.async_copy(gb_hbm.at[qv.at[pl.ds(j * 80, 80)]], rowsb, sem).wait()
        pltpu.sync_copy(rowsb, outb_hbm.at[pl.ds(base + j * 80, 80), :])
        return 0
    lax.fori_loop(0, QPT // 80, chunk, 0)


def _sc_qgather(g_tables, gb_tables, qidx):
    k = pl.kernel(
        _qgather_body,
        out_type=[jax.ShapeDtypeStruct((TSTEPS * NQ, WA), jnp.float32),
                  jax.ShapeDtypeStruct((TSTEPS * NQ, WB), jnp.float32)],
        mesh=_mesh(),
        scratch_types=[
            pltpu.VMEM((QPT,), jnp.int32),
            pltpu.VMEM((80, WA), jnp.float32),
            pltpu.VMEM((80, WB), jnp.float32),
            pltpu.SemaphoreType.DMA,
        ],
        compiler_params=pltpu.CompilerParams(needs_layout_passes=False, use_tc_tiling_on_sc=False),
    )
    return k(g_tables, gb_tables, qidx)


_BLK = 640
_NROW = TSTEPS * NTP
_NBLK = _NROW // _BLK


def _tca_body(degp, emb, dinv, x1, x1b):
    deg = jnp.sum(degp[...], axis=0) + 1.0
    dv = lax.rsqrt(deg)[:, None]
    dinv[...] = dv
    x1[...] = dv * emb[...]
    x1b[...] = jnp.concatenate(
        [dv, jnp.zeros((_BLK, WB - 1), jnp.float32)], axis=-1)


def _tc_prep1(deg_partials, embp8):
    return pl.pallas_call(
        _tca_body,
        grid=(_NBLK,),
        in_specs=[
            pl.BlockSpec((NW, _BLK), lambda j: (0, j)),
            pl.BlockSpec((_BLK, NINP), lambda j: (j % (NTP // _BLK), 0)),
        ],
        out_specs=[
            pl.BlockSpec((_BLK, 1), lambda j: (j, 0)),
            pl.BlockSpec((_BLK, WA), lambda j: (j, 0)),
            pl.BlockSpec((_BLK, WB), lambda j: (j, 0)),
        ],
        out_shape=[
            jax.ShapeDtypeStruct((_NROW, 1), jnp.float32),
            jax.ShapeDtypeStruct((_NROW, WA), jnp.float32),
            jax.ShapeDtypeStruct((_NROW, WB), jnp.float32),
        ],
    )(deg_partials, embp8)


def _tcb_body(s1p, s1pb, x1, x1b, dinv, x2, rs):
    s = s1p[0] + s1p[1] + x1[...]
    p = dinv[...] * s
    x2[...] = dinv[...] * p
    sb = s1pb[0] + s1pb[1] + x1b[...]
    rs[...] = dinv[...] * sb[:, 0:1]


def _tc_prep2(s1_partials, s1b_partials, x1, x1b, dinv):
    return pl.pallas_call(
        _tcb_body,
        grid=(_NBLK,),
        in_specs=[
            pl.BlockSpec((NC, _BLK, WA), lambda j: (0, j, 0)),
            pl.BlockSpec((NC, _BLK, WB), lambda j: (0, j, 0)),
            pl.BlockSpec((_BLK, WA), lambda j: (j, 0)),
            pl.BlockSpec((_BLK, WB), lambda j: (j, 0)),
            pl.BlockSpec((_BLK, 1), lambda j: (j, 0)),
        ],
        out_specs=[
            pl.BlockSpec((_BLK, WA), lambda j: (j, 0)),
            pl.BlockSpec((_BLK, 1), lambda j: (j, 0)),
        ],
        out_shape=[
            jax.ShapeDtypeStruct((_NROW, WA), jnp.float32),
            jax.ShapeDtypeStruct((_NROW, 1), jnp.float32),
        ],
    )(s1_partials, s1b_partials, x1, x1b, dinv)


def _tcc_body(s2p, x2, dinv, rs, g, gb):
    s = s2p[0] + s2p[1] + x2[...]
    g[...] = dinv[...] * s
    gb[...] = jnp.concatenate(
        [rs[...], jnp.zeros((_BLK, WB - 1), jnp.float32)], axis=-1)


def _tc_prep3(s2_partials, x2, dinv, rs):
    return pl.pallas_call(
        _tcc_body,
        grid=(_NBLK,),
        in_specs=[
            pl.BlockSpec((NC, _BLK, WA), lambda j: (0, j, 0)),
            pl.BlockSpec((_BLK, WA), lambda j: (j, 0)),
            pl.BlockSpec((_BLK, 1), lambda j: (j, 0)),
            pl.BlockSpec((_BLK, 1), lambda j: (j, 0)),
        ],
        out_specs=[
            pl.BlockSpec((_BLK, WA), lambda j: (j, 0)),
            pl.BlockSpec((_BLK, WB), lambda j: (j, 0)),
        ],
        out_shape=[
            jax.ShapeDtypeStruct((_NROW, WA), jnp.float32),
            jax.ShapeDtypeStruct((_NROW, WB), jnp.float32),
        ],
    )(s2_partials, x2, dinv, rs)


def _tcw_body(wg1, wg2, bg1, w12, rvec):
    w12[...] = jnp.dot(wg1[...], wg2[...],
                       preferred_element_type=jnp.float32)
    rvec[...] = jnp.dot(bg1[...], wg2[...],
                        preferred_element_type=jnp.float32)


def _tc_w(Wg1, Wg2, bg1):
    return pl.pallas_call(
        _tcw_body,
        out_shape=[
            jax.ShapeDtypeStruct((NINP, NINP), jnp.float32),
            jax.ShapeDtypeStruct((1, NINP), jnp.float32),
        ],
    )(Wg1, Wg2, bg1.reshape(1, 2 * NINP))


def _layer_norm(x, g, b):
    m = jnp.mean(x, axis=-1, keepdims=True)
    v = jnp.mean((x - m) ** 2, axis=-1, keepdims=True)
    return (x - m) * lax.rsqrt(v + 1e-5) * g + b


def _tcd_body(raw_ref, rawb_ref, ts_ref, seq_ref, tem_ref, pos_ref, w12_ref, rvec_ref,
              bg2_ref, wq_ref, bq_ref, wk_ref, bk_ref, wv_ref, bv_ref,
              wo_ref, bo_ref, l1g_ref, l1b_ref, l2g_ref, l2b_ref,
              wf1_ref, bf1_ref, wf2_ref, bf2_ref, out_ref):
    raw = raw_ref[...][:, 0]
    rawb = rawb_ref[...][:, 0]
    w12 = w12_ref[...]
    dyu = (jnp.dot(raw.reshape(TSTEPS * L, NINP), w12,
                   preferred_element_type=jnp.float32).reshape(TSTEPS, L, NINP)
           + rawb[..., 0][:, :, None] * rvec_ref[...][None]
           + bg2_ref[...][None])
    ts = ts_ref[0, 0]
    onehot = (ts[:, None] == lax.broadcasted_iota(jnp.int32, (L, TSTEPS), 1))
    tem = jnp.dot(onehot.astype(jnp.float32), tem_ref[...],
                  preferred_element_type=jnp.float32)
    temperature = 128 ** 0.5 + 1e-06
    affine = jnp.sum(tem[None] * dyu, axis=-1) / temperature
    affine = affine - jnp.max(affine, axis=1, keepdims=True)
    ea = jnp.exp(affine)
    alpha = ea / jnp.sum(ea, axis=1, keepdims=True)
    dyemb = jnp.sum(alpha[:, :, None] * dyu, axis=0)
    x = jnp.concatenate([dyemb, pos_ref[...]], axis=-1)

    q = jnp.dot(x, wq_ref[...], preferred_element_type=jnp.float32) + bq_ref[...]
    k = jnp.dot(x, wk_ref[...], preferred_element_type=jnp.float32) + bk_ref[...]
    v = jnp.dot(x, wv_ref[...], preferred_element_type=jnp.float32) + bv_ref[...]
    kmask = (seq_ref[0, 0] == PAD)[None, :]
    heads = []
    scale = 1.0 / (float(HD) ** 0.5)
    for h in range(NHEADS):
        sl = slice(h * HD, (h + 1) * HD)
        qh, kh, vh = q[:, sl], k[:, sl], v[:, sl]
        sc = lax.dot_general(qh, kh, (((1,), (1,)), ((), ())),
                             preferred_element_type=jnp.float32) * scale
        sc = jnp.where(kmask, -1e9, sc)
        sc = sc - jnp.max(sc, axis=-1, keepdims=True)
        es = jnp.exp(sc)
        attn = es / jnp.sum(es, axis=-1, keepdims=True)
        heads.append(jnp.dot(attn, vh, preferred_element_type=jnp.float32))
    o = jnp.concatenate(heads, axis=-1)
    hh = _layer_norm(x + jnp.dot(o, wo_ref[...],
                                 preferred_element_type=jnp.float32)
                     + bo_ref[...], l1g_ref[...], l1b_ref[...])
    ff = jnp.dot(jnp.maximum(jnp.dot(hh, wf1_ref[...],
                                     preferred_element_type=jnp.float32)
                             + bf1_ref[...], 0.0), wf2_ref[...],
                 preferred_element_type=jnp.float32) + bf2_ref[...]
    out_ref[...] = _layer_norm(hh + ff, l2g_ref[...], l2b_ref[...])[None]


def _tc_head(dyu_raw, dyu_rawb, dyemb_ts, seq, time_emb, pos200, w12, rvec,
             bg2, Wq, bq, Wk, bk, Wv, bv, Wo, bo, ln1g, ln1b, ln2g, ln2b,
             Wf1, bf1, Wf2, bf2):
    row = lambda a: a.reshape(1, -1)

    def full(a):
        nd = a.ndim
        return pl.BlockSpec(a.shape, lambda b, _nd=nd: (0,) * _nd)
    ins = [dyu_raw.reshape(TSTEPS, BATCH, L, WA),
           dyu_rawb.reshape(TSTEPS, BATCH, L, WB)]
    specs = [pl.BlockSpec((TSTEPS, 1, L, WA), lambda b: (0, b, 0, 0)),
             pl.BlockSpec((TSTEPS, 1, L, WB), lambda b: (0, b, 0, 0))]
    for a in (dyemb_ts, seq):
        ins.append(a.reshape(BATCH, 1, L))
        specs.append(pl.BlockSpec((1, 1, L), lambda b: (b, 0, 0)))
    for a in (time_emb, pos200, w12, rvec, row(bg2), Wq, row(bq), Wk, row(bk),
              Wv, row(bv), Wo, row(bo), row(ln1g), row(ln1b), row(ln2g),
              row(ln2b), Wf1, row(bf1), Wf2, row(bf2)):
        ins.append(a)
        specs.append(full(a))
    return pl.pallas_call(
        _tcd_body,
        grid=(BATCH,),
        in_specs=specs,
        out_specs=pl.BlockSpec((1, L, D), lambda b: (b, 0, 0)),
        out_shape=jax.ShapeDtypeStruct((BATCH, L, D), jnp.float32),
    )(*ins)


def _tce_body(att_ref, wout_ref, bout_ref, seq_ref, out_ref):
    mm = (jnp.dot(att_ref[0], wout_ref[...],
                  preferred_element_type=jnp.float32) + bout_ref[...])
    viota = lax.broadcasted_iota(jnp.int32, (L, NTOKEN), 1)
    hit = (seq_ref[0, 0][:, None] == viota).astype(jnp.int32)
    sh = 1
    while sh < L:
        z = jnp.zeros((sh, NTOKEN), jnp.int32)
        hit = hit | jnp.concatenate([z, hit[:L - sh]], axis=0)
        sh *= 2
    masked = (hit > 0) | (viota == 0)
    out_ref[...] = jnp.where(masked, -jnp.inf, mm)


def _tc_proj(att_out, Wout, bout, seq):
    return pl.pallas_call(
        _tce_body,
        grid=(BATCH,),
        in_specs=[
            pl.BlockSpec((1, L, D), lambda b: (b, 0, 0)),
            pl.BlockSpec((D, NTOKEN), lambda b: (0, 0)),
            pl.BlockSpec((1, NTOKEN), lambda b: (0, 0)),
            pl.BlockSpec((1, 1, L), lambda b: (b, 0, 0)),
        ],
        out_specs=pl.BlockSpec((L, NTOKEN), lambda b: (b, 0)),
        out_shape=jax.ShapeDtypeStruct((BATCH * L, NTOKEN), jnp.float32),
    )(att_out, Wout, bout.reshape(1, NTOKEN), seq.reshape(BATCH, 1, L))


def _dyemb_ts(input_timestamp):
    ts = input_timestamp[:, :-1]
    Bz, Ls = ts.shape
    pad = (-Ls) % 5
    padded = jnp.pad(ts, ((0, 0), (0, pad)))
    nb = (Ls + pad) // 5
    blocks = padded.reshape(Bz, nb, 5)
    la = blocks.max(axis=(0, 2))
    active = jnp.cumprod((la >= 1).astype(jnp.int32)) > 0
    res_index = jnp.minimum(la, TSTEPS - 1)
    vals = jnp.where(active, res_index, 0).astype(jnp.int32)
    dy = jnp.broadcast_to(vals[None, :, None], (Bz, nb, 5)).reshape(Bz, nb * 5)
    return dy[:, :Ls]


def kernel(input, input_timestamp, diffusion_edge_index, emb, Wg1, bg1, Wg2,
           bg2, pos_emb, time_emb, Wq, bq, Wk, bk, Wv, bv, Wo, bo, ln1g, ln1b,
           ln2g, ln2b, Wf1, bf1, Wf2, bf2, Wout, bout):
    seq = input[:, :-1]
    dyemb_ts = _dyemb_ts(input_timestamp)
    src_flat = diffusion_edge_index[:, 0, :].reshape(-1)
    dst_flat = diffusion_edge_index[:, 1, :].reshape(-1)
    embp = jnp.pad(emb, ((0, NTP - NTOKEN), (0, 0)))

    deg_partials = _sc_degree(dst_flat)
    dinv, x1, x1b = _tc_prep1(deg_partials, embp)
    s1p, s1pb = _sc_hop(x1, src_flat, dst_flat, xb_tables=x1b)
    x2, rs = _tc_prep2(s1p, s1pb, x1, x1b, dinv)
    qflag = jnp.zeros((NTP,), jnp.int32).at[seq.reshape(-1)].set(1)
    s2p = _sc_hop(x2, src_flat, dst_flat, qflag=qflag)
    g, gb = _tc_prep3(s2p, x2, dinv, rs)

    qidx = (jnp.arange(TSTEPS, dtype=jnp.int32)[:, None] * NTP
            + seq.reshape(-1)[None, :]).reshape(-1)
    dyu_raw, dyu_rawb = _sc_qgather(g, gb, qidx)

    w12, rvec = _tc_w(Wg1, Wg2, bg1)
    pos200 = pos_emb[:L]
    att_out = _tc_head(dyu_raw, dyu_rawb, dyemb_ts, seq, time_emb, pos200,
                       w12, rvec, bg2, Wq, bq, Wk, bk, Wv, bv, Wo, bo,
                       ln1g, ln1b, ln2g, ln2b, Wf1, bf1, Wf2, bf2)
    return _tc_proj(att_out, Wout, bout, seq)


# speedup vs baseline: 23.4003x; 1.1156x over previous
import functools

import jax
import jax.numpy as jnp
from jax import lax
from jax.experimental import pallas as pl
from jax.experimental.pallas import tpu as pltpu
from jax.experimental.pallas import tpu_sc as plsc

PAD = 0
NTOKEN = 10000
NINP = 128
POS_DIM = 8
TSTEPS = 8
NHEADS = 8
BATCH = 8
SEQ = 201
NEDGES = 320000

NTP = 10240
WA = 128
WB = 16
NC, NS, NW = 2, 16, 32
EPT = NEDGES // NW
CH = 64
L = SEQ - 1
D = NINP + POS_DIM
HD = D // NHEADS
NQ = BATCH * L
QPT = NQ * TSTEPS // NW
RPT = NTP // NS

_mesh = functools.partial(plsc.VectorSubcoreMesh,
                          core_axis_name="c", subcore_axis_name="s")


def _deg_body(dst_hbm, out_hbm, dstr, hist):
    cid = lax.axis_index("c")
    sid = lax.axis_index("s")
    wid = cid * NS + sid
    zv = jnp.zeros((16,), jnp.float32)
    ones = jnp.ones((16,), jnp.float32)

    def zero_step(i, _):
        hist[i, :] = zv
        return 0
    lax.fori_loop(0, TSTEPS * NTP // 16, zero_step, 0)

    def t_step(t, _):
        pltpu.sync_copy(dst_hbm.at[pl.ds(t * NEDGES + wid * EPT, EPT)], dstr)

        def e_step(i, _):
            idx = dstr[pl.ds(i * 16, 16)] + t * NTP
            plsc.addupdate_scatter(hist, [idx >> 4, idx & 15], ones)
            return 0
        lax.fori_loop(0, EPT // 16, e_step, 0)
        return 0
    lax.fori_loop(0, TSTEPS, t_step, 0)
    pltpu.sync_copy(hist, out_hbm.at[wid])


def _sc_degree(dst_flat):
    k = pl.kernel(
        _deg_body,
        out_type=jax.ShapeDtypeStruct((NW, TSTEPS * NTP // 16, 16), jnp.float32),
        mesh=_mesh(),
        scratch_types=[
            pltpu.VMEM((EPT,), jnp.int32),
            pltpu.VMEM((TSTEPS * NTP // 16, 16), jnp.float32),
        ],
        compiler_params=pltpu.CompilerParams(needs_layout_passes=False, use_tc_tiling_on_sc=False),
    )
    return k(dst_flat).reshape(NW, TSTEPS * NTP)


HB = EPT // 2
NCH2 = (HB + CH - 1) // CH
SPAD = NCH2 * CH


def _hop_body(has_aux, filt, chv, *args):
    CH = chv
    NCH2 = (HB + CH - 1) // CH
    qflag_hbm = qflag_v = None
    rowsb0 = rowsb1 = semb0 = semb1 = None
    if has_aux and filt:
        (x_hbm, xb_hbm, src_hbm, dst_hbm, qflag_hbm, out_hbm, outb_hbm,
         srcr, dstr, dsts, rows0, rows1, rowsb0, rowsb1, qflag_v,
         acc, accb, sem0, sem1, semb0, semb1) = args
    elif has_aux:
        (x_hbm, xb_hbm, src_hbm, dst_hbm, out_hbm, outb_hbm, srcr, dstr, dsts,
         rows0, rows1, rowsb0, rowsb1, acc, accb, sem0, sem1, semb0, semb1) = args
    elif filt:
        (x_hbm, src_hbm, dst_hbm, qflag_hbm, out_hbm, srcr, dstr, dsts,
         rows0, rows1, qflag_v, acc, sem0, sem1) = args
    else:
        (x_hbm, src_hbm, dst_hbm, out_hbm, srcr, dstr, dsts,
         rows0, rows1, acc, sem0, sem1) = args
    cid = lax.axis_index("c")
    sid = lax.axis_index("s")
    wid = cid * NS + sid
    zv = jnp.zeros((16,), jnp.float32)
    iota = jnp.arange(16, dtype=jnp.int32)
    if filt:
        pltpu.sync_copy(qflag_hbm, qflag_v)

    def start_gather(j, rows, sem, rowsb, semb):
        pltpu.async_copy(x_hbm.at[srcr.at[pl.ds(j * CH, CH)]], rows, sem)
        if has_aux:
            pltpu.async_copy(xb_hbm.at[srcr.at[pl.ds(j * CH, CH)]], rowsb, semb)

    def t_step(t, _):
        def z_step(i, _):
            j = i // (WA // 16)
            k = i % (WA // 16)
            rows0[j, pl.ds(k * 16, 16)] = zv
            return 0
        lax.fori_loop(0, CH * (WA // 16), z_step, 0)

        def clr(m, _):
            pltpu.sync_copy(rows0, acc.at[pl.ds(sid * RPT + m * CH, CH)])
            return 0
        lax.fori_loop(0, RPT // CH, clr, 0)
        if has_aux:
            def zb_step(i, _):
                rowsb0[i, :] = zv
                return 0
            lax.fori_loop(0, CH, zb_step, 0)

            def clrb(m, _):
                pltpu.sync_copy(rowsb0, accb.at[pl.ds(sid * RPT + m * CH, CH)])
                return 0
            lax.fori_loop(0, RPT // CH, clrb, 0)

        gpad = t * NTP + NTOKEN + sid * 8
        spad = NTOKEN + sid * 8 + cid

        def batch(b2, _):
            ebase = t * NEDGES + wid * EPT + b2 * HB
            pltpu.sync_copy(src_hbm.at[pl.ds(ebase, HB)],
                            srcr.at[pl.ds(0, HB)])
            pltpu.sync_copy(dst_hbm.at[pl.ds(ebase, HB)],
                            dstr.at[pl.ds(0, HB)])

            if filt:
                def cstage(i, cur):
                    pos = i * 16
                    valid = (pos + iota) < HB
                    sv = srcr[pl.ds(pos, 16)]
                    dv = dstr[pl.ds(pos, 16)]
                    dv0 = jnp.where(valid, dv, 0)
                    w = plsc.load_gather(qflag_v, [dv0 >> 5])
                    keep = valid & (((w >> (dv0 & 31)) & 1) > 0)
                    plsc.store_compressed(srcr.at[pl.ds(cur, 16)],
                                          sv + t * NTP, mask=keep)
                    plsc.store_compressed(dstr.at[pl.ds(cur, 16)],
                                          dv0, mask=keep)
                    return cur + plsc.all_reduce_population_count(keep)[0]
                count = lax.fori_loop(0, (HB + 15) // 16, cstage, 0)
                nch = (count + CH - 1) // CH

                def pfill(i, _):
                    g = (count // 16 + i) * 16
                    m = (g + iota) >= count
                    sv = srcr[pl.ds(g, 16)]
                    dv = dstr[pl.ds(g, 16)]
                    srcr[pl.ds(g, 16)] = jnp.where(m, gpad, sv)
                    dstr[pl.ds(g, 16)] = jnp.where(m, spad, dv)
                    return 0
                lax.fori_loop(0, (CH // 16) + 1, pfill, 0)

                def move(i, _):
                    j = i // (CH // 16)
                    k = i % (CH // 16)
                    dsts[j, pl.ds(k * 16, 16)] = dstr[pl.ds(i * 16, 16)]
                    return 0
                lax.fori_loop(0, nch * (CH // 16), move, 0)
            else:
                nch = NCH2

                def stage(i, _):
                    j = i // (CH // 16)
                    k = i % (CH // 16)
                    pos = i * 16
                    valid = (pos + iota) < HB
                    sv = srcr[pl.ds(pos, 16)]
                    dv = dstr[pl.ds(pos, 16)]
                    srcr[pl.ds(pos, 16)] = jnp.where(valid, sv + t * NTP, gpad)
                    dsts[j, pl.ds(k * 16, 16)] = jnp.where(valid, dv, spad)
                    return 0
                lax.fori_loop(0, NCH2 * (CH // 16), stage, 0)

            lax.cond(nch > 0,
                     lambda: start_gather(0, rows0, sem0, rowsb0, semb0),
                     lambda: None)

            def chunk(j, _):
                def phase(rows, sem, rowsb, semb, orows, osem, orowsb, osemb):
                    pltpu.make_async_copy(
                        x_hbm.at[srcr.at[pl.ds(j * CH, CH)]], rows, sem).wait()
                    if has_aux:
                        pltpu.make_async_copy(
                            xb_hbm.at[srcr.at[pl.ds(j * CH, CH)]],
                            rowsb, semb).wait()
                    lax.cond(j + 1 < nch,
                             lambda: start_gather(
                                 jnp.minimum(j + 1, NCH2 - 1),
                                 orows, osem, orowsb, osemb),
                             lambda: None)
                    pltpu.sync_copy(rows, acc.at[dsts.at[j]], add=True)
                    if has_aux:
                        pltpu.sync_copy(rowsb, accb.at[dsts.at[j]], add=True)
                lax.cond(j % 2 == 0,
                         lambda: phase(rows0, sem0, rowsb0, semb0,
                                       rows1, sem1, rowsb1, semb1),
                         lambda: phase(rows1, sem1, rowsb1, semb1,
                                       rows0, sem0, rowsb0, semb0))
                return 0
            lax.fori_loop(0, nch, chunk, 0)
            return 0
        lax.fori_loop(0, 2, batch, 0)

        plsc.subcore_barrier()

        def wout(m, _):
            r0 = sid * RPT + m * CH
            pltpu.sync_copy(acc.at[pl.ds(r0, CH)],
                            out_hbm.at[cid, t, pl.ds(r0, CH), :])
            return 0
        lax.fori_loop(0, RPT // CH, wout, 0)
        if has_aux:
            def woutb(m, _):
                r0 = sid * RPT + m * CH
                pltpu.sync_copy(accb.at[pl.ds(r0, CH)],
                                outb_hbm.at[cid, t, pl.ds(r0, CH), :])
                return 0
            lax.fori_loop(0, RPT // CH, woutb, 0)

        plsc.subcore_barrier()
        return 0
    lax.fori_loop(0, TSTEPS, t_step, 0)


def _sc_hop(x_tables, src_flat, dst_flat, xb_tables=None, qflag=None, chv=64):
    has_aux = xb_tables is not None
    filt = qflag is not None
    nch2 = (HB + chv - 1) // chv
    spad = nch2 * chv
    out_type = [jax.ShapeDtypeStruct((NC, TSTEPS, NTP, WA), jnp.float32)]
    scratch = [
        pltpu.VMEM((spad,), jnp.int32),
        pltpu.VMEM((spad,), jnp.int32),
        pltpu.VMEM((nch2, chv), jnp.int32),
        pltpu.VMEM((chv, WA), jnp.float32),
        pltpu.VMEM((chv, WA), jnp.float32),
    ]
    if has_aux:
        out_type.append(jax.ShapeDtypeStruct((NC, TSTEPS, NTP, WB), jnp.float32))
        scratch.append(pltpu.VMEM((chv, WB), jnp.float32))
        scratch.append(pltpu.VMEM((chv, WB), jnp.float32))
    if filt:
        scratch.append(pltpu.VMEM((NTP // 32,), jnp.int32))
    scratch.append(pltpu.VMEM_SHARED((NTP, WA), jnp.float32))
    if has_aux:
        scratch.append(pltpu.VMEM_SHARED((NTP, WB), jnp.float32))
    nsem = 4 if has_aux else 2
    for _ in range(nsem):
        scratch.append(pltpu.SemaphoreType.DMA)
    k = pl.kernel(
        functools.partial(_hop_body, has_aux, filt, chv),
        out_type=out_type if has_aux else out_type[0],
        name=f"sc_hop_aux{int(has_aux)}_f{int(filt)}_c{chv}",
        mesh=_mesh(),
        scratch_types=scratch,
        compiler_params=pltpu.CompilerParams(needs_layout_passes=False,
                                             use_tc_tiling_on_sc=False),
    )
    args = [x_tables]
    if has_aux:
        args.append(xb_tables)
    args += [src_flat, dst_flat]
    if filt:
        args.append(qflag)
    if has_aux:
        s, sb = k(*args)
        return (s.reshape(NC, TSTEPS * NTP, WA),
                sb.reshape(NC, TSTEPS * NTP, WB))
    s = k(*args)
    return s.reshape(NC, TSTEPS * NTP, WA)


def _qgather_body(g_hbm, gb_hbm, qidx_hbm, out_hbm, outb_hbm,
                  qv, rows, rowsb, sem):
    cid = lax.axis_index("c")
    sid = lax.axis_index("s")
    wid = cid * NS + sid
    base = wid * QPT
    pltpu.sync_copy(qidx_hbm.at[pl.ds(base, QPT)], qv)

    def chunk(j, _):
        pltpu.async_copy(g_hbm.at[qv.at[pl.ds(j * 80, 80)]], rows, sem).wait()
        pltpu.sync_copy(rows, out_hbm.at[pl.ds(base + j * 80, 80), :])
        pltpu.async_copy(gb_hbm.at[qv.at[pl.ds(j * 80, 80)]], rowsb, sem).wait()
        pltpu.sync_copy(rowsb, outb_hbm.at[pl.ds(base + j * 80, 80), :])
        return 0
    lax.fori_loop(0, QPT // 80, chunk, 0)


def _sc_qgather(g_tables, gb_tables, qidx):
    k = pl.kernel(
        _qgather_body,
        out_type=[jax.ShapeDtypeStruct((TSTEPS * NQ, WA), jnp.float32),
                  jax.ShapeDtypeStruct((TSTEPS * NQ, WB), jnp.float32)],
        mesh=_mesh(),
        scratch_types=[
            pltpu.VMEM((QPT,), jnp.int32),
            pltpu.VMEM((80, WA), jnp.float32),
            pltpu.VMEM((80, WB), jnp.float32),
            pltpu.SemaphoreType.DMA,
        ],
        compiler_params=pltpu.CompilerParams(needs_layout_passes=False, use_tc_tiling_on_sc=False),
    )
    return k(g_tables, gb_tables, qidx)


_BLK = 640
_NROW = TSTEPS * NTP
_NBLK = _NROW // _BLK


def _tca_body(degp, emb, dinv, x1, x1b):
    deg = jnp.sum(degp[...], axis=0) + 1.0
    dv = lax.rsqrt(deg)[:, None]
    dinv[...] = dv
    x1[...] = dv * emb[...]
    x1b[...] = jnp.concatenate(
        [dv, jnp.zeros((_BLK, WB - 1), jnp.float32)], axis=-1)


def _tc_prep1(deg_partials, embp8):
    return pl.pallas_call(
        _tca_body,
        grid=(_NBLK,),
        in_specs=[
            pl.BlockSpec((NW, _BLK), lambda j: (0, j)),
            pl.BlockSpec((_BLK, NINP), lambda j: (j % (NTP // _BLK), 0)),
        ],
        out_specs=[
            pl.BlockSpec((_BLK, 1), lambda j: (j, 0)),
            pl.BlockSpec((_BLK, WA), lambda j: (j, 0)),
            pl.BlockSpec((_BLK, WB), lambda j: (j, 0)),
        ],
        out_shape=[
            jax.ShapeDtypeStruct((_NROW, 1), jnp.float32),
            jax.ShapeDtypeStruct((_NROW, WA), jnp.float32),
            jax.ShapeDtypeStruct((_NROW, WB), jnp.float32),
        ],
    )(deg_partials, embp8)


def _tcb_body(s1p, x1, dinv, x2):
    s = s1p[0] + s1p[1] + x1[...]
    x2[...] = dinv[...] * dinv[...] * s


def _tc_prep2(s1_partials, x1, dinv):
    return pl.pallas_call(
        _tcb_body,
        grid=(_NBLK,),
        in_specs=[
            pl.BlockSpec((NC, _BLK, WA), lambda j: (0, j, 0)),
            pl.BlockSpec((_BLK, WA), lambda j: (j, 0)),
            pl.BlockSpec((_BLK, 1), lambda j: (j, 0)),
        ],
        out_specs=pl.BlockSpec((_BLK, WA), lambda j: (j, 0)),
        out_shape=jax.ShapeDtypeStruct((_NROW, WA), jnp.float32),
    )(s1_partials, x1, dinv)


def _tcc_body(s2p, s2pb, x2, x1b, dinv, g, gb):
    s = s2p[0] + s2p[1] + x2[...]
    g[...] = dinv[...] * s
    sb = s2pb[0] + s2pb[1] + x1b[...]
    rs = dinv[...] * sb[:, 0:1]
    gb[...] = jnp.concatenate(
        [rs, jnp.zeros((_BLK, WB - 1), jnp.float32)], axis=-1)


def _tc_prep3(s2_partials, s2b_partials, x2, x1b, dinv):
    return pl.pallas_call(
        _tcc_body,
        grid=(_NBLK,),
        in_specs=[
            pl.BlockSpec((NC, _BLK, WA), lambda j: (0, j, 0)),
            pl.BlockSpec((NC, _BLK, WB), lambda j: (0, j, 0)),
            pl.BlockSpec((_BLK, WA), lambda j: (j, 0)),
            pl.BlockSpec((_BLK, WB), lambda j: (j, 0)),
            pl.BlockSpec((_BLK, 1), lambda j: (j, 0)),
        ],
        out_specs=[
            pl.BlockSpec((_BLK, WA), lambda j: (j, 0)),
            pl.BlockSpec((_BLK, WB), lambda j: (j, 0)),
        ],
        out_shape=[
            jax.ShapeDtypeStruct((_NROW, WA), jnp.float32),
            jax.ShapeDtypeStruct((_NROW, WB), jnp.float32),
        ],
    )(s2_partials, s2b_partials, x2, x1b, dinv)


def _tcw_body(wg1, wg2, bg1, w12, rvec):
    w12[...] = jnp.dot(wg1[...], wg2[...],
                       preferred_element_type=jnp.float32)
    rvec[...] = jnp.dot(bg1[...], wg2[...],
                        preferred_element_type=jnp.float32)


def _tc_w(Wg1, Wg2, bg1):
    return pl.pallas_call(
        _tcw_body,
        out_shape=[
            jax.ShapeDtypeStruct((NINP, NINP), jnp.float32),
            jax.ShapeDtypeStruct((1, NINP), jnp.float32),
        ],
    )(Wg1, Wg2, bg1.reshape(1, 2 * NINP))


def _layer_norm(x, g, b):
    m = jnp.mean(x, axis=-1, keepdims=True)
    v = jnp.mean((x - m) ** 2, axis=-1, keepdims=True)
    return (x - m) * lax.rsqrt(v + 1e-5) * g + b


def _tcd_body(raw_ref, rawb_ref, ts_ref, seq_ref, tem_ref, pos_ref, w12_ref, rvec_ref,
              bg2_ref, wq_ref, bq_ref, wk_ref, bk_ref, wv_ref, bv_ref,
              wo_ref, bo_ref, l1g_ref, l1b_ref, l2g_ref, l2b_ref,
              wf1_ref, bf1_ref, wf2_ref, bf2_ref, out_ref):
    raw = raw_ref[...][:, 0]
    rawb = rawb_ref[...][:, 0]
    w12 = w12_ref[...]
    dyu = (jnp.dot(raw.reshape(TSTEPS * L, NINP), w12,
                   preferred_element_type=jnp.float32).reshape(TSTEPS, L, NINP)
           + rawb[..., 0][:, :, None] * rvec_ref[...][None]
           + bg2_ref[...][None])
    ts = ts_ref[0, 0]
    onehot = (ts[:, None] == lax.broadcasted_iota(jnp.int32, (L, TSTEPS), 1))
    tem = jnp.dot(onehot.astype(jnp.float32), tem_ref[...],
                  preferred_element_type=jnp.float32)
    temperature = 128 ** 0.5 + 1e-06
    affine = jnp.sum(tem[None] * dyu, axis=-1) / temperature
    affine = affine - jnp.max(affine, axis=1, keepdims=True)
    ea = jnp.exp(affine)
    alpha = ea / jnp.sum(ea, axis=1, keepdims=True)
    dyemb = jnp.sum(alpha[:, :, None] * dyu, axis=0)
    x = jnp.concatenate([dyemb, pos_ref[...]], axis=-1)

    q = jnp.dot(x, wq_ref[...], preferred_element_type=jnp.float32) + bq_ref[...]
    k = jnp.dot(x, wk_ref[...], preferred_element_type=jnp.float32) + bk_ref[...]
    v = jnp.dot(x, wv_ref[...], preferred_element_type=jnp.float32) + bv_ref[...]
    kmask = (seq_ref[0, 0] == PAD)[None, :]
    heads = []
    scale = 1.0 / (float(HD) ** 0.5)
    for h in range(NHEADS):
        sl = slice(h * HD, (h + 1) * HD)
        qh, kh, vh = q[:, sl], k[:, sl], v[:, sl]
        sc = lax.dot_general(qh, kh, (((1,), (1,)), ((), ())),
                             preferred_element_type=jnp.float32) * scale
        sc = jnp.where(kmask, -1e9, sc)
        sc = sc - jnp.max(sc, axis=-1, keepdims=True)
        es = jnp.exp(sc)
        attn = es / jnp.sum(es, axis=-1, keepdims=True)
        heads.append(jnp.dot(attn, vh, preferred_element_type=jnp.float32))
    o = jnp.concatenate(heads, axis=-1)
    hh = _layer_norm(x + jnp.dot(o, wo_ref[...],
                                 preferred_element_type=jnp.float32)
                     + bo_ref[...], l1g_ref[...], l1b_ref[...])
    ff = jnp.dot(jnp.maximum(jnp.dot(hh, wf1_ref[...],
                                     preferred_element_type=jnp.float32)
                             + bf1_ref[...], 0.0), wf2_ref[...],
                 preferred_element_type=jnp.float32) + bf2_ref[...]
    out_ref[...] = _layer_norm(hh + ff, l2g_ref[...], l2b_ref[...])[None]


def _tc_head(dyu_raw, dyu_rawb, dyemb_ts, seq, time_emb, pos200, w12, rvec,
             bg2, Wq, bq, Wk, bk, Wv, bv, Wo, bo, ln1g, ln1b, ln2g, ln2b,
             Wf1, bf1, Wf2, bf2):
    row = lambda a: a.reshape(1, -1)

    def full(a):
        nd = a.ndim
        return pl.BlockSpec(a.shape, lambda b, _nd=nd: (0,) * _nd)
    ins = [dyu_raw.reshape(TSTEPS, BATCH, L, WA),
           dyu_rawb.reshape(TSTEPS, BATCH, L, WB)]
    specs = [pl.BlockSpec((TSTEPS, 1, L, WA), lambda b: (0, b, 0, 0)),
             pl.BlockSpec((TSTEPS, 1, L, WB), lambda b: (0, b, 0, 0))]
    for a in (dyemb_ts, seq):
        ins.append(a.reshape(BATCH, 1, L))
        specs.append(pl.BlockSpec((1, 1, L), lambda b: (b, 0, 0)))
    for a in (time_emb, pos200, w12, rvec, row(bg2), Wq, row(bq), Wk, row(bk),
              Wv, row(bv), Wo, row(bo), row(ln1g), row(ln1b), row(ln2g),
              row(ln2b), Wf1, row(bf1), Wf2, row(bf2)):
        ins.append(a)
        specs.append(full(a))
    return pl.pallas_call(
        _tcd_body,
        grid=(BATCH,),
        in_specs=specs,
        out_specs=pl.BlockSpec((1, L, D), lambda b: (b, 0, 0)),
        out_shape=jax.ShapeDtypeStruct((BATCH, L, D), jnp.float32),
    )(*ins)


def _tce_body(att_ref, wout_ref, bout_ref, seq_ref, out_ref):
    mm = (jnp.dot(att_ref[0], wout_ref[...],
                  preferred_element_type=jnp.float32) + bout_ref[...])
    viota = lax.broadcasted_iota(jnp.int32, (L, NTOKEN), 1)
    hit = (seq_ref[0, 0][:, None] == viota).astype(jnp.int32)
    sh = 1
    while sh < L:
        z = jnp.zeros((sh, NTOKEN), jnp.int32)
        hit = hit | jnp.concatenate([z, hit[:L - sh]], axis=0)
        sh *= 2
    masked = (hit > 0) | (viota == 0)
    out_ref[...] = jnp.where(masked, -jnp.inf, mm)


def _tc_proj(att_out, Wout, bout, seq):
    return pl.pallas_call(
        _tce_body,
        grid=(BATCH,),
        in_specs=[
            pl.BlockSpec((1, L, D), lambda b: (b, 0, 0)),
            pl.BlockSpec((D, NTOKEN), lambda b: (0, 0)),
            pl.BlockSpec((1, NTOKEN), lambda b: (0, 0)),
            pl.BlockSpec((1, 1, L), lambda b: (b, 0, 0)),
        ],
        out_specs=pl.BlockSpec((L, NTOKEN), lambda b: (b, 0)),
        out_shape=jax.ShapeDtypeStruct((BATCH * L, NTOKEN), jnp.float32),
    )(att_out, Wout, bout.reshape(1, NTOKEN), seq.reshape(BATCH, 1, L))


def _dyemb_ts(input_timestamp):
    ts = input_timestamp[:, :-1]
    Bz, Ls = ts.shape
    pad = (-Ls) % 5
    padded = jnp.pad(ts, ((0, 0), (0, pad)))
    nb = (Ls + pad) // 5
    blocks = padded.reshape(Bz, nb, 5)
    la = blocks.max(axis=(0, 2))
    active = jnp.cumprod((la >= 1).astype(jnp.int32)) > 0
    res_index = jnp.minimum(la, TSTEPS - 1)
    vals = jnp.where(active, res_index, 0).astype(jnp.int32)
    dy = jnp.broadcast_to(vals[None, :, None], (Bz, nb, 5)).reshape(Bz, nb * 5)
    return dy[:, :Ls]


def kernel(input, input_timestamp, diffusion_edge_index, emb, Wg1, bg1, Wg2,
           bg2, pos_emb, time_emb, Wq, bq, Wk, bk, Wv, bv, Wo, bo, ln1g, ln1b,
           ln2g, ln2b, Wf1, bf1, Wf2, bf2, Wout, bout):
    seq = input[:, :-1]
    dyemb_ts = _dyemb_ts(input_timestamp)
    src_flat = diffusion_edge_index[:, 0, :].reshape(-1)
    dst_flat = diffusion_edge_index[:, 1, :].reshape(-1)
    embp = jnp.pad(emb, ((0, NTP - NTOKEN), (0, 0)))

    deg_partials = _sc_degree(dst_flat)
    dinv, x1, x1b = _tc_prep1(deg_partials, embp)
    s1p = _sc_hop(x1, src_flat, dst_flat, chv=128)
    x2 = _tc_prep2(s1p, x1, dinv)
    qflag1 = jnp.zeros((NTP,), jnp.int32).at[seq.reshape(-1)].set(1)
    qflag = jnp.sum(qflag1.reshape(NTP // 32, 32)
                    << jnp.arange(32, dtype=jnp.int32)[None, :], axis=1,
                    dtype=jnp.int32)
    s2p, s2pb = _sc_hop(x2, src_flat, dst_flat, xb_tables=x1b, qflag=qflag,
                        chv=64)
    g, gb = _tc_prep3(s2p, s2pb, x2, x1b, dinv)

    qidx = (jnp.arange(TSTEPS, dtype=jnp.int32)[:, None] * NTP
            + seq.reshape(-1)[None, :]).reshape(-1)
    dyu_raw, dyu_rawb = _sc_qgather(g, gb, qidx)

    w12, rvec = _tc_w(Wg1, Wg2, bg1)
    pos200 = pos_emb[:L]
    att_out = _tc_head(dyu_raw, dyu_rawb, dyemb_ts, seq, time_emb, pos200,
                       w12, rvec, bg2, Wq, bq, Wk, bk, Wv, bv, Wo, bo,
                       ln1g, ln1b, ln2g, ln2b, Wf1, bf1, Wf2, bf2)
    return _tc_proj(att_out, Wout, bout, seq)
